# Initial kernel scaffold; baseline (speedup 1.0000x reference)
import jax
import jax.numpy as jnp
from jax.experimental import pallas as pl


def kernel(x, edge_index, edge_weight, W_lin, edge_table, W_heads, a_src, a_dst):
    raise NotImplementedError("write your pallas kernel here")


# speedup vs baseline: 21.3112x; 21.3112x over previous
import functools

import jax
import jax.numpy as jnp
from jax import lax
from jax.experimental import pallas as pl
from jax.experimental.pallas import tpu as pltpu
from jax.experimental.pallas import tpu_sc as plsc

N = 10000
E = 320000
D = 128
NH = 3
NEG_SLOPE = 0.2

NC = 2
NS = 16
L = 16
NW = NC * NS
EPT = E // NW
BB = 400
BC = 80

_mesh = plsc.VectorSubcoreMesh(core_axis_name="c", subcore_axis_name="s")
_sc_params = pltpu.CompilerParams(needs_layout_passes=False)


def _dense_body(x_ref, wlin_ref, wh_ref, asrc_ref, adst_ref, h_ref, asd_ref):
    x1 = jnp.dot(x_ref[...], wlin_ref[...], preferred_element_type=jnp.float32)
    cols = []
    for h in range(NH):
        hh = jnp.dot(x1, wh_ref[h, :, :], preferred_element_type=jnp.float32)
        h_ref[:, pl.ds(h * D, D)] = hh
        a_s = jnp.dot(hh, asrc_ref[h, :].reshape(D, 1),
                      preferred_element_type=jnp.float32)
        a_d = jnp.dot(hh, adst_ref[h, :].reshape(D, 1),
                      preferred_element_type=jnp.float32)
        cols.append((a_s, a_d))
    asd_ref[...] = jnp.concatenate(
        [cols[0][0], cols[1][0], cols[2][0],
         cols[0][1], cols[1][1], cols[2][1]], axis=1)


def _dense(x, w_lin, w_heads, a_src, a_dst):
    return pl.pallas_call(
        _dense_body,
        out_shape=[jax.ShapeDtypeStruct((N, NH * D), jnp.float32),
                   jax.ShapeDtypeStruct((N, 6), jnp.float32)],
    )(x, w_lin, w_heads, a_src, a_dst)


@functools.partial(
    pl.kernel, mesh=_mesh,
    compiler_params=_sc_params,
    out_type=[jax.ShapeDtypeStruct((E * 3,), jnp.float32),
              jax.ShapeDtypeStruct((NW, 1, N * 3), jnp.float32)],
    scratch_types=[
        pltpu.VMEM((N * 6,), jnp.float32),
        pltpu.VMEM((N * 3,), jnp.float32),
        pltpu.VMEM((BB,), jnp.int32),
        pltpu.VMEM((BB,), jnp.int32),
        pltpu.VMEM((BB * 3,), jnp.float32),
    ])
def _phase_b(src_hbm, dst_hbm, asd_hbm, p_hbm, den_hbm,
             asd_v, den_v, src_v, dst_v, p_v):
    cid = lax.axis_index("c")
    sid = lax.axis_index("s")
    wid = sid * NC + cid
    base = wid * EPT

    pltpu.sync_copy(asd_hbm, asd_v)

    zeros = jnp.zeros((L,), jnp.float32)

    def zero_body(i, carry):
        den_v[pl.ds(i * L, L)] = zeros
        return carry
    lax.fori_loop(0, (N * 3) // L, zero_body, 0)

    iota = lax.iota(jnp.int32, L)

    def blk_body(b, carry):
        off = base + b * BB
        pltpu.sync_copy(src_hbm.at[pl.ds(off, BB)], src_v)
        pltpu.sync_copy(dst_hbm.at[pl.ds(off, BB)], dst_v)

        def grp_body(i, c2):
            sv = src_v[pl.ds(i * L, L)]
            dv = dst_v[pl.ds(i * L, L)]
            s6 = sv * 6
            d6 = dv * 6
            d3 = dv * 3
            lane = iota + i * L
            for h in range(NH):
                va = plsc.load_gather(asd_v, [s6 + h])
                vb = plsc.load_gather(asd_v, [d6 + (3 + h)])
                e = va + vb
                e = jnp.where(e >= 0.0, e, e * NEG_SLOPE)
                p = jnp.exp(e)
                plsc.addupdate_scatter(den_v, [d3 + h], p)
                plsc.store_scatter(p_v, [lane * 3 + h], p)
            return c2
        lax.fori_loop(0, BB // L, grp_body, 0)
        pltpu.sync_copy(p_v, p_hbm.at[pl.ds(off * 3, BB * 3)])
        return carry
    lax.fori_loop(0, EPT // BB, blk_body, 0)

    pltpu.sync_copy(den_v, den_hbm.at[wid, 0])


def _reduce_body(den_ref, rec_ref):
    s = jnp.sum(den_ref[...], axis=0)
    rec_ref[...] = 1.0 / (s + 1e-16)


def _reduce_den(den_parts):
    return pl.pallas_call(
        _reduce_body,
        out_shape=jax.ShapeDtypeStruct((N * 3,), jnp.float32),
    )(den_parts.reshape(NW, N * 3))


@functools.partial(
    pl.kernel, mesh=_mesh,
    compiler_params=_sc_params,
    out_type=jax.ShapeDtypeStruct((E * 3,), jnp.float32),
    scratch_types=[
        pltpu.VMEM((N * 3,), jnp.float32),
        pltpu.VMEM((BB,), jnp.int32),
        pltpu.VMEM((BB * 3,), jnp.float32),
        pltpu.VMEM((BB * 3,), jnp.float32),
    ])
def _normalize(dst_hbm, p_hbm, rec_hbm, al_hbm, rec_v, dst_v, p_v, al_v):
    cid = lax.axis_index("c")
    sid = lax.axis_index("s")
    wid = sid * NC + cid
    base = wid * EPT

    pltpu.sync_copy(rec_hbm, rec_v)
    iota = lax.iota(jnp.int32, L)

    def blk_body(b, carry):
        off = base + b * BB
        pltpu.sync_copy(dst_hbm.at[pl.ds(off, BB)], dst_v)
        pltpu.sync_copy(p_hbm.at[pl.ds(off * 3, BB * 3)], p_v)

        def grp_body(k, c2):
            j = iota + k * L
            pv = p_v[pl.ds(k * L, L)]
            dv = plsc.load_gather(dst_v, [j // 3])
            rv = plsc.load_gather(rec_v, [dv * 3 + j % 3])
            al_v[pl.ds(k * L, L)] = pv * rv
            return c2
        lax.fori_loop(0, (BB * 3) // L, grp_body, 0)
        pltpu.sync_copy(al_v, al_hbm.at[pl.ds(off * 3, BB * 3)])
        return carry
    lax.fori_loop(0, EPT // BB, blk_body, 0)


@functools.partial(
    pl.kernel, mesh=_mesh,
    compiler_params=_sc_params,
    out_type=jax.ShapeDtypeStruct((NC, N, D), jnp.float32),
    scratch_types=[
        pltpu.VMEM((BC,), jnp.int32),
        pltpu.VMEM((BC,), jnp.int32),
        pltpu.VMEM((BC,), jnp.int32),
        pltpu.VMEM((BC, NH * D), jnp.float32),
        pltpu.VMEM((BC * 3 + L,), jnp.float32),
        pltpu.VMEM((BC, D), jnp.float32),
        pltpu.VMEM_SHARED((N, D), jnp.float32),
        pltpu.SemaphoreType.DMA,
        pltpu.SemaphoreType.DMA,
    ])
def _phase_c(src_hbm, dst_hbm, w_hbm, h_hbm, tab_hbm, al_hbm,
             out_hbm,
             src_v, dst_v, w_v, hrow_v, al_v, msg_v,
             acc_sh, sem_h, sem_e):
    cid = lax.axis_index("c")
    sid = lax.axis_index("s")
    wid = sid * NC + cid
    base = wid * EPT

    zeros = jnp.zeros((L,), jnp.float32)

    def zero_body(i, carry):
        r = i // (D // L)
        c = i % (D // L)
        msg_v[r, pl.ds(c * L, L)] = zeros
        return carry
    lax.fori_loop(0, BC * (D // L), zero_body, 0)
    nchunks = N // BC
    for j in range((nchunks + NS - 1) // NS):
        chunk = sid + j * NS

        @pl.when(chunk < nchunks)
        def _():
            pltpu.sync_copy(msg_v, acc_sh.at[pl.ds(chunk * BC, BC)])
    plsc.subcore_barrier()

    iota = lax.iota(jnp.int32, L)

    def blk_body(b, carry):
        off = base + b * BC
        pltpu.sync_copy(src_hbm.at[pl.ds(off, BC)], src_v)
        pltpu.sync_copy(dst_hbm.at[pl.ds(off, BC)], dst_v)
        pltpu.sync_copy(w_hbm.at[pl.ds(off, BC)], w_v)
        pltpu.sync_copy(al_hbm.at[pl.ds(off * 3, BC * 3)],
                        al_v.at[pl.ds(0, BC * 3)])
        cp_h = pltpu.async_copy(h_hbm.at[src_v], hrow_v, sem_h)
        cp_e = pltpu.async_copy(tab_hbm.at[w_v], msg_v, sem_e)
        cp_h.wait()
        cp_e.wait()

        def edge_body(e, c2):
            av = al_v[pl.ds(3 * e, L)]
            a0 = av[0]
            a1 = av[1]
            a2 = av[2]
            for db in range(D // L):
                h0 = hrow_v[e, pl.ds(db * L, L)]
                h1 = hrow_v[e, pl.ds(D + db * L, L)]
                h2 = hrow_v[e, pl.ds(2 * D + db * L, L)]
                s = a0 * h0 + a1 * h1 + a2 * h2
                m = s * msg_v[e, pl.ds(db * L, L)]
                msg_v[e, pl.ds(db * L, L)] = m
            return c2
        lax.fori_loop(0, BC, edge_body, 0)

        pltpu.sync_copy(msg_v, acc_sh.at[dst_v], add=True)
        return carry
    lax.fori_loop(0, EPT // BC, blk_body, 0)

    plsc.subcore_barrier()
    r1 = 632

    @pl.when(sid < NS - 1)
    def _():
        pltpu.sync_copy(acc_sh.at[pl.ds(sid * r1, r1)],
                        out_hbm.at[cid, pl.ds(sid * r1, r1)])

    @pl.when(sid == NS - 1)
    def _():
        pltpu.sync_copy(acc_sh.at[pl.ds((NS - 1) * r1, N - (NS - 1) * r1)],
                        out_hbm.at[cid, pl.ds((NS - 1) * r1,
                                              N - (NS - 1) * r1)])


def _combine_body(parts_ref, out_ref):
    out_ref[...] = (parts_ref[0] + parts_ref[1]) * (1.0 / NH)


def _combine(parts):
    return pl.pallas_call(
        _combine_body,
        out_shape=jax.ShapeDtypeStruct((N, D), jnp.float32),
    )(parts)


def kernel(x, edge_index, edge_weight, W_lin, edge_table, W_heads, a_src,
           a_dst):
    src = edge_index[0].astype(jnp.int32)
    dst = edge_index[1].astype(jnp.int32)
    w = edge_weight.astype(jnp.int32)
    h_cat, asd = _dense(x, W_lin, W_heads, a_src, a_dst)
    p, den_parts = _phase_b(src, dst, asd.reshape(-1))
    rec = _reduce_den(den_parts)
    alpha = _normalize(dst, p, rec)
    parts = _phase_c(src, dst, w, h_cat, edge_table, alpha)
    return _combine(parts)


# speedup vs baseline: 33.9670x; 1.5939x over previous
import functools

import jax
import jax.numpy as jnp
import numpy as np
from jax import lax
from jax.experimental import pallas as pl
from jax.experimental.pallas import tpu as pltpu
from jax.experimental.pallas import tpu_sc as plsc

N = 10000
E = 320000
D = 128
NH = 3
NEG_SLOPE = 0.2
V = 22754

NC = 2
NS = 16
L = 16
NW = NC * NS
EPT = E // NW
BB = 400
BC = 64
NBLK = EPT // BC
TAIL = EPT - NBLK * BC
HROW = 256

_mesh = plsc.VectorSubcoreMesh(core_axis_name="c", subcore_axis_name="s")
_sc_params = pltpu.CompilerParams(needs_layout_passes=False)

_PINV = np.zeros((D, D), np.float32)
for _g in range(D // 32):
    for _i in range(16):
        _PINV[_g * 32 + _i, _g * 32 + 2 * _i] = 1.0
        _PINV[_g * 32 + 16 + _i, _g * 32 + 2 * _i + 1] = 1.0


def _dense_body(x_ref, wlin_ref, wh_ref, asrc_ref, adst_ref, pinv_ref,
                h_ref, asd_ref):
    x1 = jnp.dot(x_ref[...], wlin_ref[...], preferred_element_type=jnp.float32)
    pinv = pinv_ref[...]
    cols = []
    for h in range(NH):
        wh = wh_ref[h, :, :]
        wp = jnp.dot(wh, pinv, preferred_element_type=jnp.float32)
        hp = jnp.dot(x1, wp, preferred_element_type=jnp.float32)
        h_ref[:, pl.ds(h * D, D)] = hp.astype(jnp.bfloat16)
        if h == 0:
            h_ref[:, pl.ds(NH * D, D)] = jnp.zeros((N, D), jnp.bfloat16)
        vs = jnp.dot(wh, asrc_ref[h, :].reshape(D, 1),
                     preferred_element_type=jnp.float32)
        vd = jnp.dot(wh, adst_ref[h, :].reshape(D, 1),
                     preferred_element_type=jnp.float32)
        cols.append((jnp.dot(x1, vs, preferred_element_type=jnp.float32),
                     jnp.dot(x1, vd, preferred_element_type=jnp.float32)))
    asd_ref[...] = jnp.concatenate(
        [cols[0][0], cols[1][0], cols[2][0],
         cols[0][1], cols[1][1], cols[2][1]], axis=1)


def _dense(x, w_lin, w_heads, a_src, a_dst, pinv):
    return pl.pallas_call(
        _dense_body,
        out_shape=[jax.ShapeDtypeStruct((N, 2 * HROW), jnp.bfloat16),
                   jax.ShapeDtypeStruct((N, 6), jnp.float32)],
    )(x, w_lin, w_heads, a_src, a_dst, pinv)


NBB = EPT // BB


@functools.partial(
    pl.kernel, mesh=_mesh,
    compiler_params=_sc_params,
    out_type=[jax.ShapeDtypeStruct((E * 8,), jnp.int32),
              jax.ShapeDtypeStruct((NW, 1, N * 3), jnp.float32)],
    scratch_types=[
        pltpu.VMEM((N * 6,), jnp.float32),
        pltpu.VMEM((N * 3,), jnp.float32),
        pltpu.VMEM((BB,), jnp.int32),
        pltpu.VMEM((BB,), jnp.int32),
        pltpu.VMEM((BB,), jnp.int32),
        pltpu.VMEM((BB,), jnp.int32),
        pltpu.VMEM((BB,), jnp.int32),
        pltpu.VMEM((BB,), jnp.int32),
        pltpu.VMEM((BB * 8,), jnp.int32),
        pltpu.VMEM((BB * 8,), jnp.int32),
        pltpu.SemaphoreType.DMA,
        pltpu.SemaphoreType.DMA,
        pltpu.SemaphoreType.DMA,
        pltpu.SemaphoreType.DMA,
    ])
def _phase_b(src_hbm, dst_hbm, w_hbm, asd_hbm, pk_hbm, den_hbm,
             asd_v, den_v, sb0, db0, wb0, sb1, db1, wb1, pk0, pk1,
             semr0, semr1, semw0, semw1):
    cid = lax.axis_index("c")
    sid = lax.axis_index("s")
    wid = sid * NC + cid
    base = wid * EPT

    pltpu.sync_copy(asd_hbm, asd_v)

    zeros = jnp.zeros((L,), jnp.float32)

    def zero_body(i, carry):
        den_v[pl.ds(i * L, L)] = zeros
        return carry
    lax.fori_loop(0, (N * 3) // L, zero_body, 0)

    iota = lax.iota(jnp.int32, L)

    def rd_copies(k, sb, db, wb, sem):
        off = base + k * BB
        return (pltpu.make_async_copy(src_hbm.at[pl.ds(off, BB)], sb, sem),
                pltpu.make_async_copy(dst_hbm.at[pl.ds(off, BB)], db, sem),
                pltpu.make_async_copy(w_hbm.at[pl.ds(off, BB)], wb, sem))

    def wr_copy(k, pk, sem):
        return pltpu.make_async_copy(
            pk, pk_hbm.at[pl.ds((base + k * BB) * 8, BB * 8)], sem)

    def compute(sb, db, wb, pk):
        def grp_body(i, c2):
            sv = sb[pl.ds(i * L, L)]
            dv = db[pl.ds(i * L, L)]
            wv = wb[pl.ds(i * L, L)]
            s6 = sv * 6
            d6 = dv * 6
            d3 = dv * 3
            l8 = (iota + i * L) * 8
            plsc.store_scatter(pk, [l8], sv)
            plsc.store_scatter(pk, [l8 + 1], dv)
            plsc.store_scatter(pk, [l8 + 2], wv)
            for h in range(NH):
                va = plsc.load_gather(asd_v, [s6 + h])
                vb = plsc.load_gather(asd_v, [d6 + (3 + h)])
                e = va + vb
                e = jnp.where(e >= 0.0, e, e * NEG_SLOPE)
                p = jnp.exp(e)
                plsc.addupdate_scatter(den_v, [d3 + h], p)
                plsc.store_scatter(pk, [l8 + 3 + h],
                                   plsc.bitcast(p, jnp.int32))
            return c2
        lax.fori_loop(0, BB // L, grp_body, 0)

    bufs0 = (sb0, db0, wb0, pk0, semr0, semw0)
    bufs1 = (sb1, db1, wb1, pk1, semr1, semw1)

    def blk(k, cur, nxt):
        sb_c, db_c, wb_c, pk_c, semr_c, semw_c = cur
        sb_n, db_n, wb_n, pk_n, semr_n, semw_n = nxt

        @pl.when(k + 1 <= NBB - 1)
        def _():
            for c in rd_copies(k + 1, sb_n, db_n, wb_n, semr_n):
                c.start()
        for c in rd_copies(k, sb_c, db_c, wb_c, semr_c):
            c.wait()

        @pl.when(k >= 2)
        def _():
            wr_copy(k - 2, pk_c, semw_c).wait()
        compute(sb_c, db_c, wb_c, pk_c)
        wr_copy(k, pk_c, semw_c).start()

    def pair_body(j, carry):
        blk(2 * j, bufs0, bufs1)
        blk(2 * j + 1, bufs1, bufs0)
        return carry
    for c in rd_copies(0, sb0, db0, wb0, semr0):
        c.start()
    lax.fori_loop(0, NBB // 2, pair_body, 0)
    blk(NBB - 1, bufs0, bufs1)

    wr_copy(NBB - 2, pk1, semw1).wait()
    wr_copy(NBB - 1, pk0, semw0).wait()

    pltpu.sync_copy(den_v, den_hbm.at[wid, 0])


def _reduce_body(den_ref, rec_ref):
    s = jnp.sum(den_ref[...], axis=0)
    rec_ref[...] = 1.0 / (s + 1e-16)


def _reduce_den(den_parts):
    return pl.pallas_call(
        _reduce_body,
        out_shape=jax.ShapeDtypeStruct((N * 3,), jnp.float32),
    )(den_parts.reshape(NW, N * 3))


BN = 2000
NBN = EPT // BN


@functools.partial(
    pl.kernel, mesh=_mesh,
    compiler_params=_sc_params,
    out_type=jax.ShapeDtypeStruct((E * 8,), jnp.int32),
    scratch_types=[
        pltpu.VMEM((N * 3,), jnp.float32),
        pltpu.VMEM((BN * 8,), jnp.int32),
        pltpu.VMEM((BN * 8,), jnp.int32),
        pltpu.SemaphoreType.DMA,
        pltpu.SemaphoreType.DMA,
        pltpu.SemaphoreType.DMA,
        pltpu.SemaphoreType.DMA,
    ])
def _normalize(pkin_hbm, rec_hbm, pkout_hbm,
               rec_v, pk0, pk1, semr0, semr1, semw0, semw1):
    cid = lax.axis_index("c")
    sid = lax.axis_index("s")
    wid = sid * NC + cid
    base = wid * EPT

    pltpu.sync_copy(rec_hbm, rec_v)
    iota = lax.iota(jnp.int32, L)

    def rd_copy(k, pk, sem):
        return pltpu.make_async_copy(
            pkin_hbm.at[pl.ds((base + k * BN) * 8, BN * 8)], pk, sem)

    def wr_copy(k, pk, sem):
        return pltpu.make_async_copy(
            pk, pkout_hbm.at[pl.ds((base + k * BN) * 8, BN * 8)], sem)

    def patch(pk):
        def al_body(g, c2):
            jl = iota + g * L
            e_of_j = jl // 3
            idx = e_of_j * 8 + 3 + jl % 3
            pv = plsc.bitcast(plsc.load_gather(pk, [idx]), jnp.float32)
            dv = plsc.load_gather(pk, [e_of_j * 8 + 1])
            rv = plsc.load_gather(rec_v, [dv * 3 + jl % 3])
            plsc.store_scatter(pk, [idx], plsc.bitcast(pv * rv, jnp.int32))
            return c2
        lax.fori_loop(0, (BN * 3) // L, al_body, 0)

    bufs0 = (pk0, semr0, semw0)
    bufs1 = (pk1, semr1, semw1)

    def blk(k, cur, nxt):
        pk_c, semr_c, semw_c = cur
        pk_n, semr_n, semw_n = nxt

        @pl.when(jnp.logical_and(k >= 1, k + 1 <= NBN - 1))
        def _():
            wr_copy(k - 1, pk_n, semw_n).wait()

        @pl.when(k + 1 <= NBN - 1)
        def _():
            rd_copy(k + 1, pk_n, semr_n).start()
        rd_copy(k, pk_c, semr_c).wait()
        patch(pk_c)
        wr_copy(k, pk_c, semw_c).start()

    def pair_body(j, carry):
        blk(2 * j, bufs0, bufs1)
        blk(2 * j + 1, bufs1, bufs0)
        return carry
    rd_copy(0, pk0, semr0).start()
    lax.fori_loop(0, NBN // 2, pair_body, 0)
    blk(NBN - 1, bufs0, bufs1)

    wr_copy(NBN - 2, pk1, semw1).wait()
    wr_copy(NBN - 1, pk0, semw0).wait()


@functools.partial(
    pl.kernel, mesh=_mesh,
    compiler_params=_sc_params,
    out_type=jax.ShapeDtypeStruct((NC, N, D), jnp.float32),
    scratch_types=[
        pltpu.VMEM((BC * 8 + L,), jnp.int32),
        pltpu.VMEM((BC * 8 + L,), jnp.int32),
        pltpu.VMEM((BC,), jnp.int32),
        pltpu.VMEM((BC // 2,), jnp.int32),
        pltpu.VMEM((BC // 2,), jnp.int32),
        pltpu.VMEM((BC,), jnp.int32),
        pltpu.VMEM((BC,), jnp.int32),
        pltpu.VMEM((BC // 2,), jnp.int32),
        pltpu.VMEM((BC // 2,), jnp.int32),
        pltpu.VMEM((BC,), jnp.int32),
        pltpu.VMEM((TAIL,), jnp.int32),
        pltpu.VMEM((TAIL,), jnp.int32),
        pltpu.VMEM((TAIL,), jnp.int32),
        pltpu.VMEM((BC, HROW), jnp.int32),
        pltpu.VMEM((BC, HROW), jnp.int32),
        pltpu.VMEM((BC, D), jnp.float32),
        pltpu.VMEM((BC // 2, D), jnp.float32),
        pltpu.VMEM_SHARED((N, D), jnp.float32),
        pltpu.SemaphoreType.DMA,
        pltpu.SemaphoreType.DMA,
        pltpu.SemaphoreType.DMA,
        pltpu.SemaphoreType.DMA,
        pltpu.SemaphoreType.DMA,
    ])
def _phase_c(pk_hbm, h_hbm, tab_hbm, out_hbm,
             pk0, pk1, sb0, dl0, dh0, wb0, sb1, dl1, dh1, wb1,
             sbt, dbt, wbt, hr0, hr1, ew_v, msg_v,
             acc_sh, semp0, semp1, semh0, semh1, seme):
    cid = lax.axis_index("c")
    sid = lax.axis_index("s")
    wid = sid * NC + cid
    base = wid * EPT
    HB = BC // 2

    zeros = jnp.zeros((L,), jnp.float32)

    def zero_body(i, carry):
        r = i // (D // L)
        c = i % (D // L)
        msg_v[r, pl.ds(c * L, L)] = zeros
        return carry
    lax.fori_loop(0, HB * (D // L), zero_body, 0)
    nfull = N // HB
    for j in range((nfull + NS) // NS):
        chunk = sid + j * NS

        @pl.when(chunk < nfull)
        def _():
            pltpu.sync_copy(msg_v, acc_sh.at[pl.ds(chunk * HB, HB)])

        @pl.when(chunk == nfull)
        def _():
            pltpu.sync_copy(msg_v.at[pl.ds(0, N - nfull * HB)],
                            acc_sh.at[pl.ds(nfull * HB, N - nfull * HB)])
    plsc.subcore_barrier()

    iota = lax.iota(jnp.int32, L)
    ilv = plsc.PackFormat.INTERLEAVED

    def build_idx(pk, sb, dl, dh, wb):
        for g in range(BC // L):
            lane = iota + g * L
            l8 = lane * 8
            sb[pl.ds(g * L, L)] = plsc.load_gather(pk, [l8])
            dv = plsc.load_gather(pk, [l8 + 1])
            if g < BC // (2 * L):
                dl[pl.ds(g * L, L)] = dv
            else:
                dh[pl.ds(g * L - HB, L)] = dv
            wb[pl.ds(g * L, L)] = plsc.load_gather(pk, [l8 + 2])

    def compute(pk, hr, e0, n):
        def edge_body(e, c2):
            eg = e0 + e
            av = plsc.bitcast(pk[pl.ds(eg * 8 + 3, L)], jnp.float32)
            a0 = av[0]
            a1 = av[1]
            a2 = av[2]
            for g in range(D // 32):
                h0 = plsc.bitcast(hr[eg, pl.ds(g * L, L)], jnp.bfloat16)
                h1 = plsc.bitcast(hr[eg, pl.ds(64 + g * L, L)], jnp.bfloat16)
                h2 = plsc.bitcast(hr[eg, pl.ds(128 + g * L, L)],
                                  jnp.bfloat16)
                h0a, h0b = plsc.unpack(h0, format=ilv)
                h1a, h1b = plsc.unpack(h1, format=ilv)
                h2a, h2b = plsc.unpack(h2, format=ilv)
                ea = ew_v[eg, pl.ds(g * 32, L)]
                eb = ew_v[eg, pl.ds(g * 32 + L, L)]
                ma = (a0 * h0a + a1 * h1a + a2 * h2a) * ea
                mb = (a0 * h0b + a1 * h1b + a2 * h2b) * eb
                msg_v[e, pl.ds(g * 32, L)] = ma
                msg_v[e, pl.ds(g * 32 + L, L)] = mb
            return c2
        lax.fori_loop(0, n, edge_body, 0)

    def pk_slice(k):
        return pk_hbm.at[pl.ds((base + k * BC) * 8, BC * 8)]

    pltpu.sync_copy(pk_slice(0), pk0.at[pl.ds(0, BC * 8)])
    build_idx(pk0, sb0, dl0, dh0, wb0)
    pltpu.async_copy(h_hbm.at[sb0], hr0, semh0)
    pltpu.async_copy(tab_hbm.at[wb0], ew_v, seme)
    pltpu.async_copy(pk_slice(1), pk1.at[pl.ds(0, BC * 8)], semp1)

    bufs0 = (pk0, sb0, dl0, dh0, wb0, hr0, semp0, semh0)
    bufs1 = (pk1, sb1, dl1, dh1, wb1, hr1, semp1, semh1)

    def blk(k, cur, nxt):
        pk_c, sb_c, dl_c, dh_c, wb_c, hr_c, semp_c, semh_c = cur
        pk_n, sb_n, dl_n, dh_n, wb_n, hr_n, semp_n, semh_n = nxt

        @pl.when(k + 1 <= NBLK - 1)
        def _():
            pltpu.make_async_copy(pk_slice(k + 1), pk_n.at[pl.ds(0, BC * 8)],
                                  semp_n).wait()
            build_idx(pk_n, sb_n, dl_n, dh_n, wb_n)
            pltpu.async_copy(h_hbm.at[sb_n], hr_n, semh_n)

        pltpu.make_async_copy(h_hbm.at[sb_c], hr_c, semh_c).wait()
        pltpu.make_async_copy(tab_hbm.at[wb_c], ew_v, seme).wait()
        compute(pk_c, hr_c, 0, HB)
        pltpu.sync_copy(msg_v, acc_sh.at[dl_c], add=True)
        compute(pk_c, hr_c, HB, HB)
        pltpu.sync_copy(msg_v, acc_sh.at[dh_c], add=True)

        @pl.when(k + 2 <= NBLK - 1)
        def _():
            pltpu.async_copy(pk_slice(k + 2), pk_c.at[pl.ds(0, BC * 8)],
                             semp_c)

        @pl.when(k + 1 <= NBLK - 1)
        def _():
            pltpu.async_copy(tab_hbm.at[wb_n], ew_v, seme)

    def pair_body(j, carry):
        blk(2 * j, bufs0, bufs1)
        blk(2 * j + 1, bufs1, bufs0)
        return carry
    lax.fori_loop(0, NBLK // 2, pair_body, 0)

    toff = base + NBLK * BC
    pltpu.sync_copy(pk_hbm.at[pl.ds(toff * 8, TAIL * 8)],
                    pk0.at[pl.ds(0, TAIL * 8)])
    l8 = iota * 8
    sbt[pl.ds(0, L)] = plsc.load_gather(pk0, [l8])
    dbt[pl.ds(0, L)] = plsc.load_gather(pk0, [l8 + 1])
    wbt[pl.ds(0, L)] = plsc.load_gather(pk0, [l8 + 2])
    pltpu.async_copy(h_hbm.at[sbt], hr0.at[pl.ds(0, TAIL)], semh0).wait()
    pltpu.async_copy(tab_hbm.at[wbt], ew_v.at[pl.ds(0, TAIL)], seme).wait()
    compute(pk0, hr0, 0, TAIL)
    pltpu.sync_copy(msg_v.at[pl.ds(0, TAIL)], acc_sh.at[dbt], add=True)

    plsc.subcore_barrier()
    r1 = 632

    @pl.when(sid < NS - 1)
    def _():
        pltpu.sync_copy(acc_sh.at[pl.ds(sid * r1, r1)],
                        out_hbm.at[cid, pl.ds(sid * r1, r1)])

    @pl.when(sid == NS - 1)
    def _():
        pltpu.sync_copy(acc_sh.at[pl.ds((NS - 1) * r1, N - (NS - 1) * r1)],
                        out_hbm.at[cid, pl.ds((NS - 1) * r1,
                                              N - (NS - 1) * r1)])


def _combine_body(parts_ref, out_ref):
    out_ref[...] = (parts_ref[0] + parts_ref[1]) * (1.0 / NH)


def _combine(parts):
    return pl.pallas_call(
        _combine_body,
        out_shape=jax.ShapeDtypeStruct((N, D), jnp.float32),
    )(parts)


def kernel(x, edge_index, edge_weight, W_lin, edge_table, W_heads, a_src,
           a_dst):
    src = edge_index[0].astype(jnp.int32)
    dst = edge_index[1].astype(jnp.int32)
    w = edge_weight.astype(jnp.int32)
    pinv = jnp.asarray(_PINV)
    h_bf, asd = _dense(x, W_lin, W_heads, a_src, a_dst, pinv)
    h_i32 = lax.bitcast_convert_type(h_bf.reshape(N, HROW, 2), jnp.int32)
    pk_raw, den_parts = _phase_b(src, dst, w, asd.reshape(-1))
    rec = _reduce_den(den_parts)
    packed = _normalize(pk_raw, rec)
    parts = _phase_c(packed, h_i32, edge_table)
    return _combine(parts)


# speedup vs baseline: 35.2878x; 1.0389x over previous
import functools

import jax
import jax.numpy as jnp
import numpy as np
from jax import lax
from jax.experimental import pallas as pl
from jax.experimental.pallas import tpu as pltpu
from jax.experimental.pallas import tpu_sc as plsc

N = 10000
E = 320000
D = 128
NH = 3
NEG_SLOPE = 0.2
V = 22754

NC = 2
NS = 16
L = 16
NW = NC * NS
EPT = E // NW
BB = 400
BC = 48
NBLK = EPT // BC
TAIL = EPT - NBLK * BC
HROW = 256

_mesh = plsc.VectorSubcoreMesh(core_axis_name="c", subcore_axis_name="s")
_sc_params = pltpu.CompilerParams(needs_layout_passes=False)

_PINV = np.zeros((D, D), np.float32)
for _g in range(D // 32):
    for _i in range(16):
        _PINV[_g * 32 + _i, _g * 32 + 2 * _i] = 1.0
        _PINV[_g * 32 + 16 + _i, _g * 32 + 2 * _i + 1] = 1.0


def _dense_body(x_ref, wlin_ref, wh_ref, asrc_ref, adst_ref, pinv_ref,
                h_ref, asd_ref):
    x1 = jnp.dot(x_ref[...], wlin_ref[...], preferred_element_type=jnp.float32)
    pinv = pinv_ref[...]
    cols = []
    for h in range(NH):
        wh = wh_ref[h, :, :]
        wp = jnp.dot(wh, pinv, preferred_element_type=jnp.float32)
        hp = jnp.dot(x1, wp, preferred_element_type=jnp.float32)
        h_ref[:, pl.ds(h * D, D)] = hp.astype(jnp.bfloat16)
        if h == 0:
            h_ref[:, pl.ds(NH * D, D)] = jnp.zeros((N, D), jnp.bfloat16)
        vs = jnp.dot(wh, asrc_ref[h, :].reshape(D, 1),
                     preferred_element_type=jnp.float32)
        vd = jnp.dot(wh, adst_ref[h, :].reshape(D, 1),
                     preferred_element_type=jnp.float32)
        cols.append((jnp.dot(x1, vs, preferred_element_type=jnp.float32),
                     jnp.dot(x1, vd, preferred_element_type=jnp.float32)))
    asd_ref[...] = jnp.concatenate(
        [cols[0][0], cols[1][0], cols[2][0],
         cols[0][1], cols[1][1], cols[2][1]], axis=1)


def _dense(x, w_lin, w_heads, a_src, a_dst, pinv):
    return pl.pallas_call(
        _dense_body,
        out_shape=[jax.ShapeDtypeStruct((N, 2 * HROW), jnp.bfloat16),
                   jax.ShapeDtypeStruct((N, 6), jnp.float32)],
    )(x, w_lin, w_heads, a_src, a_dst, pinv)


NBB = EPT // BB


@functools.partial(
    pl.kernel, mesh=_mesh,
    compiler_params=_sc_params,
    out_type=[jax.ShapeDtypeStruct((E * 8,), jnp.int32),
              jax.ShapeDtypeStruct((NW, 1, N * 3), jnp.float32)],
    scratch_types=[
        pltpu.VMEM((N * 6,), jnp.float32),
        pltpu.VMEM((N * 3,), jnp.float32),
        pltpu.VMEM((BB,), jnp.int32),
        pltpu.VMEM((BB,), jnp.int32),
        pltpu.VMEM((BB,), jnp.int32),
        pltpu.VMEM((BB,), jnp.int32),
        pltpu.VMEM((BB,), jnp.int32),
        pltpu.VMEM((BB,), jnp.int32),
        pltpu.VMEM((BB * 8,), jnp.int32),
        pltpu.VMEM((BB * 8,), jnp.int32),
        pltpu.SemaphoreType.DMA,
        pltpu.SemaphoreType.DMA,
        pltpu.SemaphoreType.DMA,
        pltpu.SemaphoreType.DMA,
    ])
def _phase_b(src_hbm, dst_hbm, w_hbm, asd_hbm, pk_hbm, den_hbm,
             asd_v, den_v, sb0, db0, wb0, sb1, db1, wb1, pk0, pk1,
             semr0, semr1, semw0, semw1):
    cid = lax.axis_index("c")
    sid = lax.axis_index("s")
    wid = sid * NC + cid
    base = wid * EPT

    pltpu.sync_copy(asd_hbm, asd_v)

    zeros = jnp.zeros((L,), jnp.float32)

    def zero_body(i, carry):
        den_v[pl.ds(i * L, L)] = zeros
        return carry
    lax.fori_loop(0, (N * 3) // L, zero_body, 0)

    iota = lax.iota(jnp.int32, L)

    def rd_copies(k, sb, db, wb, sem):
        off = base + k * BB
        return (pltpu.make_async_copy(src_hbm.at[pl.ds(off, BB)], sb, sem),
                pltpu.make_async_copy(dst_hbm.at[pl.ds(off, BB)], db, sem),
                pltpu.make_async_copy(w_hbm.at[pl.ds(off, BB)], wb, sem))

    def wr_copy(k, pk, sem):
        return pltpu.make_async_copy(
            pk, pk_hbm.at[pl.ds((base + k * BB) * 8, BB * 8)], sem)

    def compute(sb, db, wb, pk):
        def grp_body(i, c2):
            sv = sb[pl.ds(i * L, L)]
            dv = db[pl.ds(i * L, L)]
            wv = wb[pl.ds(i * L, L)]
            s6 = sv * 6
            d6 = dv * 6
            d3 = dv * 3
            l8 = (iota + i * L) * 8
            plsc.store_scatter(pk, [l8], sv)
            plsc.store_scatter(pk, [l8 + 1], dv)
            plsc.store_scatter(pk, [l8 + 2], wv)
            for h in range(NH):
                va = plsc.load_gather(asd_v, [s6 + h])
                vb = plsc.load_gather(asd_v, [d6 + (3 + h)])
                e = va + vb
                e = jnp.where(e >= 0.0, e, e * NEG_SLOPE)
                p = jnp.exp(e)
                plsc.addupdate_scatter(den_v, [d3 + h], p)
                plsc.store_scatter(pk, [l8 + 3 + h],
                                   plsc.bitcast(p, jnp.int32))
            return c2
        lax.fori_loop(0, BB // L, grp_body, 0)

    bufs0 = (sb0, db0, wb0, pk0, semr0, semw0)
    bufs1 = (sb1, db1, wb1, pk1, semr1, semw1)

    def blk(k, cur, nxt):
        sb_c, db_c, wb_c, pk_c, semr_c, semw_c = cur
        sb_n, db_n, wb_n, pk_n, semr_n, semw_n = nxt

        @pl.when(k + 1 <= NBB - 1)
        def _():
            for c in rd_copies(k + 1, sb_n, db_n, wb_n, semr_n):
                c.start()
        for c in rd_copies(k, sb_c, db_c, wb_c, semr_c):
            c.wait()

        @pl.when(k >= 2)
        def _():
            wr_copy(k - 2, pk_c, semw_c).wait()
        compute(sb_c, db_c, wb_c, pk_c)
        wr_copy(k, pk_c, semw_c).start()

    def pair_body(j, carry):
        blk(2 * j, bufs0, bufs1)
        blk(2 * j + 1, bufs1, bufs0)
        return carry
    for c in rd_copies(0, sb0, db0, wb0, semr0):
        c.start()
    lax.fori_loop(0, NBB // 2, pair_body, 0)
    blk(NBB - 1, bufs0, bufs1)

    wr_copy(NBB - 2, pk1, semw1).wait()
    wr_copy(NBB - 1, pk0, semw0).wait()

    pltpu.sync_copy(den_v, den_hbm.at[wid, 0])


def _reduce_body(den_ref, rec_ref):
    s = jnp.sum(den_ref[...], axis=0)
    rec_ref[...] = 1.0 / (s + 1e-16)


def _reduce_den(den_parts):
    return pl.pallas_call(
        _reduce_body,
        out_shape=jax.ShapeDtypeStruct((N * 3,), jnp.float32),
    )(den_parts.reshape(NW, N * 3))


BN = 2000
NBN = EPT // BN


@functools.partial(
    pl.kernel, mesh=_mesh,
    compiler_params=_sc_params,
    out_type=jax.ShapeDtypeStruct((E * 8,), jnp.int32),
    scratch_types=[
        pltpu.VMEM((N * 3,), jnp.float32),
        pltpu.VMEM((BN * 8,), jnp.int32),
        pltpu.VMEM((BN * 8,), jnp.int32),
        pltpu.SemaphoreType.DMA,
        pltpu.SemaphoreType.DMA,
        pltpu.SemaphoreType.DMA,
        pltpu.SemaphoreType.DMA,
    ])
def _normalize(pkin_hbm, rec_hbm, pkout_hbm,
               rec_v, pk0, pk1, semr0, semr1, semw0, semw1):
    cid = lax.axis_index("c")
    sid = lax.axis_index("s")
    wid = sid * NC + cid
    base = wid * EPT

    pltpu.sync_copy(rec_hbm, rec_v)
    iota = lax.iota(jnp.int32, L)

    def rd_copy(k, pk, sem):
        return pltpu.make_async_copy(
            pkin_hbm.at[pl.ds((base + k * BN) * 8, BN * 8)], pk, sem)

    def wr_copy(k, pk, sem):
        return pltpu.make_async_copy(
            pk, pkout_hbm.at[pl.ds((base + k * BN) * 8, BN * 8)], sem)

    def patch(pk):
        def al_body(g, c2):
            jl = iota + g * L
            e_of_j = jl // 3
            idx = e_of_j * 8 + 3 + jl % 3
            pv = plsc.bitcast(plsc.load_gather(pk, [idx]), jnp.float32)
            dv = plsc.load_gather(pk, [e_of_j * 8 + 1])
            rv = plsc.load_gather(rec_v, [dv * 3 + jl % 3])
            plsc.store_scatter(pk, [idx], plsc.bitcast(pv * rv, jnp.int32))
            return c2
        lax.fori_loop(0, (BN * 3) // L, al_body, 0)

    bufs0 = (pk0, semr0, semw0)
    bufs1 = (pk1, semr1, semw1)

    def blk(k, cur, nxt):
        pk_c, semr_c, semw_c = cur
        pk_n, semr_n, semw_n = nxt

        @pl.when(jnp.logical_and(k >= 1, k + 1 <= NBN - 1))
        def _():
            wr_copy(k - 1, pk_n, semw_n).wait()

        @pl.when(k + 1 <= NBN - 1)
        def _():
            rd_copy(k + 1, pk_n, semr_n).start()
        rd_copy(k, pk_c, semr_c).wait()
        patch(pk_c)
        wr_copy(k, pk_c, semw_c).start()

    def pair_body(j, carry):
        blk(2 * j, bufs0, bufs1)
        blk(2 * j + 1, bufs1, bufs0)
        return carry
    rd_copy(0, pk0, semr0).start()
    lax.fori_loop(0, NBN // 2, pair_body, 0)
    blk(NBN - 1, bufs0, bufs1)

    wr_copy(NBN - 2, pk1, semw1).wait()
    wr_copy(NBN - 1, pk0, semw0).wait()


@functools.partial(
    pl.kernel, mesh=_mesh,
    compiler_params=_sc_params,
    out_type=jax.ShapeDtypeStruct((NC, N, D), jnp.float32),
    scratch_types=[
        pltpu.VMEM((BC * 8 + L,), jnp.int32),
        pltpu.VMEM((BC * 8 + L,), jnp.int32),
        pltpu.VMEM((BC,), jnp.int32),
        pltpu.VMEM((BC,), jnp.int32),
        pltpu.VMEM((BC,), jnp.int32),
        pltpu.VMEM((BC,), jnp.int32),
        pltpu.VMEM((BC,), jnp.int32),
        pltpu.VMEM((BC,), jnp.int32),
        pltpu.VMEM((TAIL,), jnp.int32),
        pltpu.VMEM((TAIL,), jnp.int32),
        pltpu.VMEM((TAIL,), jnp.int32),
        pltpu.VMEM((BC, HROW), jnp.int32),
        pltpu.VMEM((BC, HROW), jnp.int32),
        pltpu.VMEM((BC, D), jnp.float32),
        pltpu.VMEM((BC, D), jnp.float32),
        pltpu.VMEM_SHARED((N, D), jnp.float32),
        pltpu.SemaphoreType.DMA,
        pltpu.SemaphoreType.DMA,
        pltpu.SemaphoreType.DMA,
        pltpu.SemaphoreType.DMA,
        pltpu.SemaphoreType.DMA,
        pltpu.SemaphoreType.DMA,
    ])
def _phase_c(pk_hbm, h_hbm, tab_hbm, out_hbm,
             pk0, pk1, sb0, db0, wb0, sb1, db1, wb1,
             sbt, dbt, wbt, hr0, hr1, ew_v, msg_v,
             acc_sh, semp0, semp1, semh0, semh1, seme, sems):
    cid = lax.axis_index("c")
    sid = lax.axis_index("s")
    wid = sid * NC + cid
    base = wid * EPT

    zeros = jnp.zeros((L,), jnp.float32)

    def zero_body(i, carry):
        r = i // (D // L)
        c = i % (D // L)
        msg_v[r, pl.ds(c * L, L)] = zeros
        return carry
    lax.fori_loop(0, BC * (D // L), zero_body, 0)
    nfull = N // BC
    for j in range((nfull + NS) // NS):
        chunk = sid + j * NS

        @pl.when(chunk < nfull)
        def _():
            pltpu.sync_copy(msg_v, acc_sh.at[pl.ds(chunk * BC, BC)])

        @pl.when(chunk == nfull)
        def _():
            pltpu.sync_copy(msg_v.at[pl.ds(0, N - nfull * BC)],
                            acc_sh.at[pl.ds(nfull * BC, N - nfull * BC)])
    plsc.subcore_barrier()

    iota = lax.iota(jnp.int32, L)
    ilv = plsc.PackFormat.INTERLEAVED

    def build_sw(pk, sb, wb):
        for g in range(BC // L):
            lane = iota + g * L
            l8 = lane * 8
            sb[pl.ds(g * L, L)] = plsc.load_gather(pk, [l8])
            wb[pl.ds(g * L, L)] = plsc.load_gather(pk, [l8 + 2])

    def build_d(pk, db):
        for g in range(BC // L):
            l8 = (iota + g * L) * 8
            db[pl.ds(g * L, L)] = plsc.load_gather(pk, [l8 + 1])

    def compute(pk, hr, n):
        def edge_body(e, c2):
            av = plsc.bitcast(pk[pl.ds(e * 8 + 3, L)], jnp.float32)
            a0 = av[0]
            a1 = av[1]
            a2 = av[2]
            for g in range(D // 32):
                h0 = plsc.bitcast(hr[e, pl.ds(g * L, L)], jnp.bfloat16)
                h1 = plsc.bitcast(hr[e, pl.ds(64 + g * L, L)], jnp.bfloat16)
                h2 = plsc.bitcast(hr[e, pl.ds(128 + g * L, L)],
                                  jnp.bfloat16)
                h0a, h0b = plsc.unpack(h0, format=ilv)
                h1a, h1b = plsc.unpack(h1, format=ilv)
                h2a, h2b = plsc.unpack(h2, format=ilv)
                ea = ew_v[e, pl.ds(g * 32, L)]
                eb = ew_v[e, pl.ds(g * 32 + L, L)]
                ma = (a0 * h0a + a1 * h1a + a2 * h2a) * ea
                mb = (a0 * h0b + a1 * h1b + a2 * h2b) * eb
                msg_v[e, pl.ds(g * 32, L)] = ma
                msg_v[e, pl.ds(g * 32 + L, L)] = mb
            return c2
        lax.fori_loop(0, n, edge_body, 0, unroll=4)

    def pk_slice(k):
        return pk_hbm.at[pl.ds((base + k * BC) * 8, BC * 8)]

    pltpu.sync_copy(pk_slice(0), pk0.at[pl.ds(0, BC * 8)])
    build_sw(pk0, sb0, wb0)
    build_d(pk0, db0)
    pltpu.async_copy(h_hbm.at[sb0], hr0, semh0)
    pltpu.async_copy(tab_hbm.at[wb0], ew_v, seme)
    pltpu.async_copy(pk_slice(1), pk1.at[pl.ds(0, BC * 8)], semp1)

    bufs0 = (pk0, sb0, db0, wb0, hr0, semp0, semh0)
    bufs1 = (pk1, sb1, db1, wb1, hr1, semp1, semh1)

    def blk(k, cur, nxt):
        pk_c, sb_c, db_c, wb_c, hr_c, semp_c, semh_c = cur
        pk_n, sb_n, db_n, wb_n, hr_n, semp_n, semh_n = nxt

        @pl.when(k + 1 <= NBLK - 1)
        def _():
            pltpu.make_async_copy(pk_slice(k + 1), pk_n.at[pl.ds(0, BC * 8)],
                                  semp_n).wait()
            build_sw(pk_n, sb_n, wb_n)
            pltpu.async_copy(h_hbm.at[sb_n], hr_n, semh_n)

        @pl.when(k >= 1)
        def _():
            pltpu.make_async_copy(msg_v, acc_sh.at[db_n], sems).wait()

        @pl.when(k + 1 <= NBLK - 1)
        def _():
            build_d(pk_n, db_n)

        pltpu.make_async_copy(h_hbm.at[sb_c], hr_c, semh_c).wait()
        pltpu.make_async_copy(tab_hbm.at[wb_c], ew_v, seme).wait()
        compute(pk_c, hr_c, BC)
        pltpu.async_copy(msg_v, acc_sh.at[db_c], sems, add=True)

        @pl.when(k + 2 <= NBLK - 1)
        def _():
            pltpu.async_copy(pk_slice(k + 2), pk_c.at[pl.ds(0, BC * 8)],
                             semp_c)

        @pl.when(k + 1 <= NBLK - 1)
        def _():
            pltpu.async_copy(tab_hbm.at[wb_n], ew_v, seme)

    def pair_body(j, carry):
        blk(2 * j, bufs0, bufs1)
        blk(2 * j + 1, bufs1, bufs0)
        return carry
    lax.fori_loop(0, NBLK // 2, pair_body, 0)

    pltpu.make_async_copy(msg_v, acc_sh.at[db1], sems).wait()

    toff = base + NBLK * BC
    pltpu.sync_copy(pk_hbm.at[pl.ds(toff * 8, TAIL * 8)],
                    pk0.at[pl.ds(0, TAIL * 8)])
    l8 = iota * 8
    sbt[pl.ds(0, L)] = plsc.load_gather(pk0, [l8])
    dbt[pl.ds(0, L)] = plsc.load_gather(pk0, [l8 + 1])
    wbt[pl.ds(0, L)] = plsc.load_gather(pk0, [l8 + 2])
    pltpu.async_copy(h_hbm.at[sbt], hr0.at[pl.ds(0, TAIL)], semh0).wait()
    pltpu.async_copy(tab_hbm.at[wbt], ew_v.at[pl.ds(0, TAIL)], seme).wait()
    compute(pk0, hr0, TAIL)
    pltpu.sync_copy(msg_v.at[pl.ds(0, TAIL)], acc_sh.at[dbt], add=True)

    plsc.subcore_barrier()
    r1 = 632

    @pl.when(sid < NS - 1)
    def _():
        pltpu.sync_copy(acc_sh.at[pl.ds(sid * r1, r1)],
                        out_hbm.at[cid, pl.ds(sid * r1, r1)])

    @pl.when(sid == NS - 1)
    def _():
        pltpu.sync_copy(acc_sh.at[pl.ds((NS - 1) * r1, N - (NS - 1) * r1)],
                        out_hbm.at[cid, pl.ds((NS - 1) * r1,
                                              N - (NS - 1) * r1)])


def _combine_body(parts_ref, out_ref):
    out_ref[...] = (parts_ref[0] + parts_ref[1]) * (1.0 / NH)


def _combine(parts):
    return pl.pallas_call(
        _combine_body,
        out_shape=jax.ShapeDtypeStruct((N, D), jnp.float32),
    )(parts)


def kernel(x, edge_index, edge_weight, W_lin, edge_table, W_heads, a_src,
           a_dst):
    src = edge_index[0].astype(jnp.int32)
    dst = edge_index[1].astype(jnp.int32)
    w = edge_weight.astype(jnp.int32)
    pinv = jnp.asarray(_PINV)
    h_bf, asd = _dense(x, W_lin, W_heads, a_src, a_dst, pinv)
    h_i32 = lax.bitcast_convert_type(h_bf.reshape(N, HROW, 2), jnp.int32)
    pk_raw, den_parts = _phase_b(src, dst, w, asd.reshape(-1))
    rec = _reduce_den(den_parts)
    packed = _normalize(pk_raw, rec)
    parts = _phase_c(packed, h_i32, edge_table)
    return _combine(parts)


# speedup vs baseline: 40.5348x; 1.1487x over previous
import functools

import jax
import jax.numpy as jnp
import numpy as np
from jax import lax
from jax.experimental import pallas as pl
from jax.experimental.pallas import tpu as pltpu
from jax.experimental.pallas import tpu_sc as plsc

N = 10000
E = 320000
D = 128
NH = 3
NEG_SLOPE = 0.2
V = 22754

NC = 2
NS = 16
L = 16
NW = NC * NS
EPT = E // NW
BB = 400
BC = 48
NBLK = EPT // BC
TAIL = EPT - NBLK * BC
HROW = 256

_mesh = plsc.VectorSubcoreMesh(core_axis_name="c", subcore_axis_name="s")
_sc_params = pltpu.CompilerParams(needs_layout_passes=False)

_SLO = np.zeros((D, D // 2), np.float32)
_SHI = np.zeros((D, D // 2), np.float32)
for _g in range(D // 32):
    for _i in range(16):
        _SLO[_g * 32 + _i, _g * 16 + _i] = 1.0
        _SHI[_g * 32 + 16 + _i, _g * 16 + _i] = 1.0


def _bf16_bits(x):
    r = x.astype(jnp.bfloat16).astype(jnp.float32)
    return lax.bitcast_convert_type(r, jnp.int32)


def _dense_body(x_ref, wlin_ref, wh_ref, asrc_ref, adst_ref, slo_ref,
                shi_ref, h_ref, asd_ref):
    x1 = jnp.dot(x_ref[...], wlin_ref[...], preferred_element_type=jnp.float32)
    slo = slo_ref[...]
    shi = shi_ref[...]
    cols = []
    words = []
    for h in range(NH):
        wh = wh_ref[h, :, :]
        hp = jnp.dot(x1, wh, preferred_element_type=jnp.float32)
        lo = jnp.dot(hp, slo, preferred_element_type=jnp.float32)
        hi = jnp.dot(hp, shi, preferred_element_type=jnp.float32)
        words.append(jnp.bitwise_or(
            lax.shift_right_logical(_bf16_bits(lo), 16),
            jnp.bitwise_and(_bf16_bits(hi), jnp.int32(-65536))))
        vs = jnp.dot(wh, asrc_ref[h, :].reshape(D, 1),
                     preferred_element_type=jnp.float32)
        vd = jnp.dot(wh, adst_ref[h, :].reshape(D, 1),
                     preferred_element_type=jnp.float32)
        cols.append((jnp.dot(x1, vs, preferred_element_type=jnp.float32),
                     jnp.dot(x1, vd, preferred_element_type=jnp.float32)))
    words.append(jnp.zeros((N, D // 2), jnp.int32))
    h_ref[...] = jnp.concatenate(words, axis=1)
    asd_ref[...] = jnp.concatenate(
        [cols[0][0], cols[1][0], cols[2][0],
         cols[0][1], cols[1][1], cols[2][1]], axis=1)


def _dense(x, w_lin, w_heads, a_src, a_dst, slo, shi):
    return pl.pallas_call(
        _dense_body,
        out_shape=[jax.ShapeDtypeStruct((N, HROW), jnp.int32),
                   jax.ShapeDtypeStruct((N, 6), jnp.float32)],
    )(x, w_lin, w_heads, a_src, a_dst, slo, shi)


NBB = EPT // BB


DEN_R = 240


@functools.partial(
    pl.kernel, mesh=_mesh,
    compiler_params=_sc_params,
    out_type=[jax.ShapeDtypeStruct((E * 8,), jnp.int32),
              jax.ShapeDtypeStruct((NC, DEN_R, 128), jnp.float32)],
    scratch_types=[
        pltpu.VMEM((N * 6,), jnp.float32),
        pltpu.VMEM((DEN_R, 128), jnp.float32),
        pltpu.VMEM((DEN_R,), jnp.int32),
        pltpu.VMEM((BB,), jnp.int32),
        pltpu.VMEM((BB,), jnp.int32),
        pltpu.VMEM((BB,), jnp.int32),
        pltpu.VMEM((BB,), jnp.int32),
        pltpu.VMEM((BB,), jnp.int32),
        pltpu.VMEM((BB,), jnp.int32),
        pltpu.VMEM((BB * 8,), jnp.int32),
        pltpu.VMEM((BB * 8,), jnp.int32),
        pltpu.VMEM_SHARED((DEN_R, 128), jnp.float32),
        pltpu.SemaphoreType.DMA,
        pltpu.SemaphoreType.DMA,
        pltpu.SemaphoreType.DMA,
        pltpu.SemaphoreType.DMA,
    ])
def _phase_b(src_hbm, dst_hbm, w_hbm, asd_hbm, pk_hbm, den_hbm,
             asd_v, den_v, rix_v, sb0, db0, wb0, sb1, db1, wb1, pk0, pk1,
             den_sh, semr0, semr1, semw0, semw1):
    cid = lax.axis_index("c")
    sid = lax.axis_index("s")
    wid = sid * NC + cid
    base = wid * EPT

    pltpu.sync_copy(asd_hbm, asd_v)

    zeros = jnp.zeros((L,), jnp.float32)
    iota = lax.iota(jnp.int32, L)

    def zero_body(i, carry):
        den_v[i // 8, pl.ds((i % 8) * L, L)] = zeros
        return carry
    lax.fori_loop(0, DEN_R * 8, zero_body, 0)

    def rix_body(g, carry):
        rix_v[pl.ds(g * L, L)] = iota + g * L
        return carry
    lax.fori_loop(0, DEN_R // L, rix_body, 0)

    pltpu.sync_copy(den_v.at[pl.ds(0, DEN_R // NS)],
                    den_sh.at[pl.ds(sid * (DEN_R // NS), DEN_R // NS)])
    plsc.subcore_barrier()

    def rd_copies(k, sb, db, wb, sem):
        off = base + k * BB
        return (pltpu.make_async_copy(src_hbm.at[pl.ds(off, BB)], sb, sem),
                pltpu.make_async_copy(dst_hbm.at[pl.ds(off, BB)], db, sem),
                pltpu.make_async_copy(w_hbm.at[pl.ds(off, BB)], wb, sem))

    def wr_copy(k, pk, sem):
        return pltpu.make_async_copy(
            pk, pk_hbm.at[pl.ds((base + k * BB) * 8, BB * 8)], sem)

    def compute(sb, db, wb, pk):
        def grp_body(i, c2):
            sv = sb[pl.ds(i * L, L)]
            dv = db[pl.ds(i * L, L)]
            wv = wb[pl.ds(i * L, L)]
            s6 = sv * 6
            d6 = dv * 6
            d3 = dv * 3
            l8 = (iota + i * L) * 8
            plsc.store_scatter(pk, [l8], sv)
            plsc.store_scatter(pk, [l8 + 1], dv)
            plsc.store_scatter(pk, [l8 + 2], wv)
            for h in range(NH):
                va = plsc.load_gather(asd_v, [s6 + h])
                vb = plsc.load_gather(asd_v, [d6 + (3 + h)])
                e = va + vb
                e = jnp.where(e >= 0.0, e, e * NEG_SLOPE)
                p = jnp.exp(e)
                idx = d3 + h
                plsc.addupdate_scatter(
                    den_v, [lax.shift_right_logical(idx, 7),
                            jnp.bitwise_and(idx, 127)], p)
                plsc.store_scatter(pk, [l8 + 3 + h],
                                   plsc.bitcast(p, jnp.int32))
            return c2
        lax.fori_loop(0, BB // L, grp_body, 0)

    bufs0 = (sb0, db0, wb0, pk0, semr0, semw0)
    bufs1 = (sb1, db1, wb1, pk1, semr1, semw1)

    def blk(k, cur, nxt):
        sb_c, db_c, wb_c, pk_c, semr_c, semw_c = cur
        sb_n, db_n, wb_n, pk_n, semr_n, semw_n = nxt

        @pl.when(k + 1 <= NBB - 1)
        def _():
            for c in rd_copies(k + 1, sb_n, db_n, wb_n, semr_n):
                c.start()
        for c in rd_copies(k, sb_c, db_c, wb_c, semr_c):
            c.wait()

        @pl.when(k >= 2)
        def _():
            wr_copy(k - 2, pk_c, semw_c).wait()
        compute(sb_c, db_c, wb_c, pk_c)
        wr_copy(k, pk_c, semw_c).start()

    def pair_body(j, carry):
        blk(2 * j, bufs0, bufs1)
        blk(2 * j + 1, bufs1, bufs0)
        return carry
    for c in rd_copies(0, sb0, db0, wb0, semr0):
        c.start()
    lax.fori_loop(0, NBB // 2, pair_body, 0)
    blk(NBB - 1, bufs0, bufs1)

    wr_copy(NBB - 2, pk1, semw1).wait()
    wr_copy(NBB - 1, pk0, semw0).wait()

    pltpu.sync_copy(den_v, den_sh.at[rix_v], add=True)
    plsc.subcore_barrier()
    rpt = DEN_R // NS
    nw16 = DEN_R // 16

    @pl.when(sid < nw16)
    def _():
        pltpu.sync_copy(den_sh.at[pl.ds(sid * 16, 16)],
                        den_hbm.at[cid, pl.ds(sid * 16, 16)])


BN = 2000
NBN = EPT // BN


@functools.partial(
    pl.kernel, mesh=_mesh,
    compiler_params=_sc_params,
    out_type=jax.ShapeDtypeStruct((E * 8,), jnp.int32),
    scratch_types=[
        pltpu.VMEM((DEN_R, 128), jnp.float32),
        pltpu.VMEM((DEN_R, 128), jnp.float32),
        pltpu.VMEM((BN * 8,), jnp.int32),
        pltpu.VMEM((BN * 8,), jnp.int32),
        pltpu.SemaphoreType.DMA,
        pltpu.SemaphoreType.DMA,
        pltpu.SemaphoreType.DMA,
        pltpu.SemaphoreType.DMA,
    ])
def _normalize(pkin_hbm, den_hbm, pkout_hbm,
               rec_v, den1_v, pk0, pk1, semr0, semr1, semw0, semw1):
    cid = lax.axis_index("c")
    sid = lax.axis_index("s")
    wid = sid * NC + cid
    base = wid * EPT

    pltpu.sync_copy(den_hbm.at[0], rec_v)
    pltpu.sync_copy(den_hbm.at[1], den1_v)
    iota = lax.iota(jnp.int32, L)

    def rec_body(i, carry):
        r = i // 8
        c = (i % 8) * L
        rec_v[r, pl.ds(c, L)] = 1.0 / (
            rec_v[r, pl.ds(c, L)] + den1_v[r, pl.ds(c, L)] + 1e-16)
        return carry
    lax.fori_loop(0, DEN_R * 8, rec_body, 0, unroll=4)

    def rd_copy(k, pk, sem):
        return pltpu.make_async_copy(
            pkin_hbm.at[pl.ds((base + k * BN) * 8, BN * 8)], pk, sem)

    def wr_copy(k, pk, sem):
        return pltpu.make_async_copy(
            pk, pkout_hbm.at[pl.ds((base + k * BN) * 8, BN * 8)], sem)

    def patch(pk):
        def al_body(g, c2):
            jl = iota + g * L
            e_of_j = jl // 3
            idx = e_of_j * 8 + 3 + jl % 3
            pv = plsc.bitcast(plsc.load_gather(pk, [idx]), jnp.float32)
            dv = plsc.load_gather(pk, [e_of_j * 8 + 1])
            ridx = dv * 3 + jl % 3
            rv = plsc.load_gather(
                rec_v, [lax.shift_right_logical(ridx, 7),
                        jnp.bitwise_and(ridx, 127)])
            plsc.store_scatter(pk, [idx], plsc.bitcast(pv * rv, jnp.int32))
            return c2
        lax.fori_loop(0, (BN * 3) // L, al_body, 0)

    bufs0 = (pk0, semr0, semw0)
    bufs1 = (pk1, semr1, semw1)

    def blk(k, cur, nxt):
        pk_c, semr_c, semw_c = cur
        pk_n, semr_n, semw_n = nxt

        @pl.when(jnp.logical_and(k >= 1, k + 1 <= NBN - 1))
        def _():
            wr_copy(k - 1, pk_n, semw_n).wait()

        @pl.when(k + 1 <= NBN - 1)
        def _():
            rd_copy(k + 1, pk_n, semr_n).start()
        rd_copy(k, pk_c, semr_c).wait()
        patch(pk_c)
        wr_copy(k, pk_c, semw_c).start()

    def pair_body(j, carry):
        blk(2 * j, bufs0, bufs1)
        blk(2 * j + 1, bufs1, bufs0)
        return carry
    rd_copy(0, pk0, semr0).start()
    lax.fori_loop(0, NBN // 2, pair_body, 0)
    blk(NBN - 1, bufs0, bufs1)

    wr_copy(NBN - 2, pk1, semw1).wait()
    wr_copy(NBN - 1, pk0, semw0).wait()


@functools.partial(
    pl.kernel, mesh=_mesh,
    compiler_params=_sc_params,
    out_type=jax.ShapeDtypeStruct((NC, N, D), jnp.float32),
    scratch_types=[
        pltpu.VMEM((BC * 8 + L,), jnp.int32),
        pltpu.VMEM((BC * 8 + L,), jnp.int32),
        pltpu.VMEM((BC,), jnp.int32),
        pltpu.VMEM((BC,), jnp.int32),
        pltpu.VMEM((BC,), jnp.int32),
        pltpu.VMEM((BC,), jnp.int32),
        pltpu.VMEM((BC,), jnp.int32),
        pltpu.VMEM((BC,), jnp.int32),
        pltpu.VMEM((TAIL,), jnp.int32),
        pltpu.VMEM((TAIL,), jnp.int32),
        pltpu.VMEM((TAIL,), jnp.int32),
        pltpu.VMEM((BC, HROW), jnp.int32),
        pltpu.VMEM((BC, HROW), jnp.int32),
        pltpu.VMEM((BC, D), jnp.float32),
        pltpu.VMEM((BC, D), jnp.float32),
        pltpu.VMEM_SHARED((N, D), jnp.float32),
        pltpu.SemaphoreType.DMA,
        pltpu.SemaphoreType.DMA,
        pltpu.SemaphoreType.DMA,
        pltpu.SemaphoreType.DMA,
        pltpu.SemaphoreType.DMA,
        pltpu.SemaphoreType.DMA,
    ])
def _phase_c(pk_hbm, h_hbm, tab_hbm, out_hbm,
             pk0, pk1, sb0, db0, wb0, sb1, db1, wb1,
             sbt, dbt, wbt, hr0, hr1, ew_v, msg_v,
             acc_sh, semp0, semp1, semh0, semh1, seme, sems):
    cid = lax.axis_index("c")
    sid = lax.axis_index("s")
    wid = sid * NC + cid
    base = wid * EPT

    zeros = jnp.zeros((L,), jnp.float32)

    def zero_body(i, carry):
        r = i // (D // L)
        c = i % (D // L)
        msg_v[r, pl.ds(c * L, L)] = zeros
        return carry
    lax.fori_loop(0, BC * (D // L), zero_body, 0)
    nfull = N // BC
    for j in range((nfull + NS) // NS):
        chunk = sid + j * NS

        @pl.when(chunk < nfull)
        def _():
            pltpu.sync_copy(msg_v, acc_sh.at[pl.ds(chunk * BC, BC)])

        @pl.when(chunk == nfull)
        def _():
            pltpu.sync_copy(msg_v.at[pl.ds(0, N - nfull * BC)],
                            acc_sh.at[pl.ds(nfull * BC, N - nfull * BC)])
    plsc.subcore_barrier()

    iota = lax.iota(jnp.int32, L)
    ilv = plsc.PackFormat.INTERLEAVED

    def build_sw(pk, sb, wb):
        for g in range(BC // L):
            lane = iota + g * L
            l8 = lane * 8
            sb[pl.ds(g * L, L)] = plsc.load_gather(pk, [l8])
            wb[pl.ds(g * L, L)] = plsc.load_gather(pk, [l8 + 2])

    def build_d(pk, db):
        for g in range(BC // L):
            l8 = (iota + g * L) * 8
            db[pl.ds(g * L, L)] = plsc.load_gather(pk, [l8 + 1])

    def compute(pk, hr, n):
        def edge_body(e, c2):
            av = plsc.bitcast(pk[pl.ds(e * 8 + 3, L)], jnp.float32)
            a0 = av[0]
            a1 = av[1]
            a2 = av[2]
            for g in range(D // 32):
                h0 = plsc.bitcast(hr[e, pl.ds(g * L, L)], jnp.bfloat16)
                h1 = plsc.bitcast(hr[e, pl.ds(64 + g * L, L)], jnp.bfloat16)
                h2 = plsc.bitcast(hr[e, pl.ds(128 + g * L, L)],
                                  jnp.bfloat16)
                h0a, h0b = plsc.unpack(h0, format=ilv)
                h1a, h1b = plsc.unpack(h1, format=ilv)
                h2a, h2b = plsc.unpack(h2, format=ilv)
                ea = ew_v[e, pl.ds(g * 32, L)]
                eb = ew_v[e, pl.ds(g * 32 + L, L)]
                ma = (a0 * h0a + a1 * h1a + a2 * h2a) * ea
                mb = (a0 * h0b + a1 * h1b + a2 * h2b) * eb
                msg_v[e, pl.ds(g * 32, L)] = ma
                msg_v[e, pl.ds(g * 32 + L, L)] = mb
            return c2
        lax.fori_loop(0, n, edge_body, 0, unroll=4)

    def pk_slice(k):
        return pk_hbm.at[pl.ds((base + k * BC) * 8, BC * 8)]

    pltpu.sync_copy(pk_slice(0), pk0.at[pl.ds(0, BC * 8)])
    build_sw(pk0, sb0, wb0)
    build_d(pk0, db0)
    pltpu.async_copy(h_hbm.at[sb0], hr0, semh0)
    pltpu.async_copy(tab_hbm.at[wb0], ew_v, seme)
    pltpu.async_copy(pk_slice(1), pk1.at[pl.ds(0, BC * 8)], semp1)

    bufs0 = (pk0, sb0, db0, wb0, hr0, semp0, semh0)
    bufs1 = (pk1, sb1, db1, wb1, hr1, semp1, semh1)

    def blk(k, cur, nxt):
        pk_c, sb_c, db_c, wb_c, hr_c, semp_c, semh_c = cur
        pk_n, sb_n, db_n, wb_n, hr_n, semp_n, semh_n = nxt

        @pl.when(k + 1 <= NBLK - 1)
        def _():
            pltpu.make_async_copy(pk_slice(k + 1), pk_n.at[pl.ds(0, BC * 8)],
                                  semp_n).wait()
            build_sw(pk_n, sb_n, wb_n)
            pltpu.async_copy(h_hbm.at[sb_n], hr_n, semh_n)

        @pl.when(k >= 1)
        def _():
            pltpu.make_async_copy(msg_v, acc_sh.at[db_n], sems).wait()

        @pl.when(k + 1 <= NBLK - 1)
        def _():
            build_d(pk_n, db_n)

        pltpu.make_async_copy(h_hbm.at[sb_c], hr_c, semh_c).wait()
        pltpu.make_async_copy(tab_hbm.at[wb_c], ew_v, seme).wait()
        compute(pk_c, hr_c, BC)
        pltpu.async_copy(msg_v, acc_sh.at[db_c], sems, add=True)

        @pl.when(k + 2 <= NBLK - 1)
        def _():
            pltpu.async_copy(pk_slice(k + 2), pk_c.at[pl.ds(0, BC * 8)],
                             semp_c)

        @pl.when(k + 1 <= NBLK - 1)
        def _():
            pltpu.async_copy(tab_hbm.at[wb_n], ew_v, seme)

    def pair_body(j, carry):
        blk(2 * j, bufs0, bufs1)
        blk(2 * j + 1, bufs1, bufs0)
        return carry
    lax.fori_loop(0, NBLK // 2, pair_body, 0)

    pltpu.make_async_copy(msg_v, acc_sh.at[db1], sems).wait()

    toff = base + NBLK * BC
    pltpu.sync_copy(pk_hbm.at[pl.ds(toff * 8, TAIL * 8)],
                    pk0.at[pl.ds(0, TAIL * 8)])
    l8 = iota * 8
    sbt[pl.ds(0, L)] = plsc.load_gather(pk0, [l8])
    dbt[pl.ds(0, L)] = plsc.load_gather(pk0, [l8 + 1])
    wbt[pl.ds(0, L)] = plsc.load_gather(pk0, [l8 + 2])
    pltpu.async_copy(h_hbm.at[sbt], hr0.at[pl.ds(0, TAIL)], semh0).wait()
    pltpu.async_copy(tab_hbm.at[wbt], ew_v.at[pl.ds(0, TAIL)], seme).wait()
    compute(pk0, hr0, TAIL)
    pltpu.sync_copy(msg_v.at[pl.ds(0, TAIL)], acc_sh.at[dbt], add=True)

    plsc.subcore_barrier()
    r1 = 632

    @pl.when(sid < NS - 1)
    def _():
        pltpu.sync_copy(acc_sh.at[pl.ds(sid * r1, r1)],
                        out_hbm.at[cid, pl.ds(sid * r1, r1)])

    @pl.when(sid == NS - 1)
    def _():
        pltpu.sync_copy(acc_sh.at[pl.ds((NS - 1) * r1, N - (NS - 1) * r1)],
                        out_hbm.at[cid, pl.ds((NS - 1) * r1,
                                              N - (NS - 1) * r1)])


def _combine_body(parts_ref, out_ref):
    out_ref[...] = (parts_ref[0] + parts_ref[1]) * (1.0 / NH)


def _combine(parts):
    return pl.pallas_call(
        _combine_body,
        out_shape=jax.ShapeDtypeStruct((N, D), jnp.float32),
    )(parts)


def kernel(x, edge_index, edge_weight, W_lin, edge_table, W_heads, a_src,
           a_dst):
    src = edge_index[0].astype(jnp.int32)
    dst = edge_index[1].astype(jnp.int32)
    w = edge_weight.astype(jnp.int32)
    slo = jnp.asarray(_SLO)
    shi = jnp.asarray(_SHI)
    h_i32, asd = _dense(x, W_lin, W_heads, a_src, a_dst, slo, shi)
    pk_raw, den_parts = _phase_b(src, dst, w, asd.reshape(-1))
    packed = _normalize(pk_raw, den_parts)
    parts = _phase_c(packed, h_i32, edge_table)
    return _combine(parts)


# speedup vs baseline: 42.1943x; 1.0409x over previous
import functools

import jax
import jax.numpy as jnp
import numpy as np
from jax import lax
from jax.experimental import pallas as pl
from jax.experimental.pallas import tpu as pltpu
from jax.experimental.pallas import tpu_sc as plsc

N = 10000
E = 320000
D = 128
NH = 3
NEG_SLOPE = 0.2
V = 22754

NC = 2
NS = 16
L = 16
NW = NC * NS
EPT = E // NW
BB = 400
BC = 64
NBLK = EPT // BC
TAIL = EPT - NBLK * BC
HROW = 192

_mesh = plsc.VectorSubcoreMesh(core_axis_name="c", subcore_axis_name="s")
_sc_params = pltpu.CompilerParams(needs_layout_passes=False)

_SLO = np.zeros((D, D // 2), np.float32)
_SHI = np.zeros((D, D // 2), np.float32)
for _g in range(D // 32):
    for _i in range(16):
        _SLO[_g * 32 + _i, _g * 16 + _i] = 1.0
        _SHI[_g * 32 + 16 + _i, _g * 16 + _i] = 1.0


def _bf16_bits(x):
    r = x.astype(jnp.bfloat16).astype(jnp.float32)
    return lax.bitcast_convert_type(r, jnp.int32)


def _dense_body(x_ref, wlin_ref, wh_ref, asrc_ref, adst_ref, slo_ref,
                shi_ref, h_ref, asd_ref):
    x1 = jnp.dot(x_ref[...], wlin_ref[...], preferred_element_type=jnp.float32)
    slo = slo_ref[...]
    shi = shi_ref[...]
    cols = []
    words = []
    for h in range(NH):
        wh = wh_ref[h, :, :]
        hp = jnp.dot(x1, wh, preferred_element_type=jnp.float32)
        lo = jnp.dot(hp, slo, preferred_element_type=jnp.float32)
        hi = jnp.dot(hp, shi, preferred_element_type=jnp.float32)
        words.append(jnp.bitwise_or(
            lax.shift_right_logical(_bf16_bits(lo), 16),
            jnp.bitwise_and(_bf16_bits(hi), jnp.int32(-65536))))
        vs = jnp.dot(wh, asrc_ref[h, :].reshape(D, 1),
                     preferred_element_type=jnp.float32)
        vd = jnp.dot(wh, adst_ref[h, :].reshape(D, 1),
                     preferred_element_type=jnp.float32)
        cols.append((jnp.dot(x1, vs, preferred_element_type=jnp.float32),
                     jnp.dot(x1, vd, preferred_element_type=jnp.float32)))
    h_ref[...] = jnp.concatenate(words, axis=1)
    asd_ref[...] = jnp.concatenate(
        [cols[0][0], cols[1][0], cols[2][0],
         cols[0][1], cols[1][1], cols[2][1]], axis=1)


def _dense(x, w_lin, w_heads, a_src, a_dst, slo, shi):
    return pl.pallas_call(
        _dense_body,
        out_shape=[jax.ShapeDtypeStruct((N, HROW), jnp.int32),
                   jax.ShapeDtypeStruct((N, 6), jnp.float32)],
    )(x, w_lin, w_heads, a_src, a_dst, slo, shi)


NBB = EPT // BB


DEN_R = 240


@functools.partial(
    pl.kernel, mesh=_mesh,
    compiler_params=_sc_params,
    out_type=[jax.ShapeDtypeStruct((E * 8,), jnp.int32),
              jax.ShapeDtypeStruct((NC, DEN_R, 128), jnp.float32)],
    scratch_types=[
        pltpu.VMEM((N * 6,), jnp.float32),
        pltpu.VMEM((DEN_R, 128), jnp.float32),
        pltpu.VMEM((DEN_R,), jnp.int32),
        pltpu.VMEM((BB,), jnp.int32),
        pltpu.VMEM((BB,), jnp.int32),
        pltpu.VMEM((BB,), jnp.int32),
        pltpu.VMEM((BB,), jnp.int32),
        pltpu.VMEM((BB,), jnp.int32),
        pltpu.VMEM((BB,), jnp.int32),
        pltpu.VMEM((BB * 8,), jnp.int32),
        pltpu.VMEM((BB * 8,), jnp.int32),
        pltpu.VMEM_SHARED((DEN_R, 128), jnp.float32),
        pltpu.SemaphoreType.DMA,
        pltpu.SemaphoreType.DMA,
        pltpu.SemaphoreType.DMA,
        pltpu.SemaphoreType.DMA,
    ])
def _phase_b(src_hbm, dst_hbm, w_hbm, asd_hbm, pk_hbm, den_hbm,
             asd_v, den_v, rix_v, sb0, db0, wb0, sb1, db1, wb1, pk0, pk1,
             den_sh, semr0, semr1, semw0, semw1):
    cid = lax.axis_index("c")
    sid = lax.axis_index("s")
    wid = sid * NC + cid
    base = wid * EPT

    pltpu.sync_copy(asd_hbm, asd_v)

    zeros = jnp.zeros((L,), jnp.float32)
    iota = lax.iota(jnp.int32, L)

    def zero_body(i, carry):
        den_v[i // 8, pl.ds((i % 8) * L, L)] = zeros
        return carry
    lax.fori_loop(0, DEN_R * 8, zero_body, 0)

    def rix_body(g, carry):
        rix_v[pl.ds(g * L, L)] = iota + g * L
        return carry
    lax.fori_loop(0, DEN_R // L, rix_body, 0)

    pltpu.sync_copy(den_v.at[pl.ds(0, DEN_R // NS)],
                    den_sh.at[pl.ds(sid * (DEN_R // NS), DEN_R // NS)])
    plsc.subcore_barrier()

    def rd_copies(k, sb, db, wb, sem):
        off = base + k * BB
        return (pltpu.make_async_copy(src_hbm.at[pl.ds(off, BB)], sb, sem),
                pltpu.make_async_copy(dst_hbm.at[pl.ds(off, BB)], db, sem),
                pltpu.make_async_copy(w_hbm.at[pl.ds(off, BB)], wb, sem))

    def wr_copy(k, pk, sem):
        return pltpu.make_async_copy(
            pk, pk_hbm.at[pl.ds((base + k * BB) * 8, BB * 8)], sem)

    def compute(sb, db, wb, pk):
        def grp_body(i, c2):
            sv = sb[pl.ds(i * L, L)]
            dv = db[pl.ds(i * L, L)]
            wv = wb[pl.ds(i * L, L)]
            s6 = sv * 6
            d6 = dv * 6
            d3 = dv * 3
            l8 = (iota + i * L) * 8
            plsc.store_scatter(pk, [l8], sv)
            plsc.store_scatter(pk, [l8 + 1], dv)
            plsc.store_scatter(pk, [l8 + 2], wv)
            for h in range(NH):
                va = plsc.load_gather(asd_v, [s6 + h])
                vb = plsc.load_gather(asd_v, [d6 + (3 + h)])
                e = va + vb
                e = jnp.where(e >= 0.0, e, e * NEG_SLOPE)
                p = jnp.exp(e)
                idx = d3 + h
                plsc.addupdate_scatter(
                    den_v, [lax.shift_right_logical(idx, 7),
                            jnp.bitwise_and(idx, 127)], p)
                plsc.store_scatter(pk, [l8 + 3 + h],
                                   plsc.bitcast(p, jnp.int32))
            return c2
        lax.fori_loop(0, BB // L, grp_body, 0)

    bufs0 = (sb0, db0, wb0, pk0, semr0, semw0)
    bufs1 = (sb1, db1, wb1, pk1, semr1, semw1)

    def blk(k, cur, nxt):
        sb_c, db_c, wb_c, pk_c, semr_c, semw_c = cur
        sb_n, db_n, wb_n, pk_n, semr_n, semw_n = nxt

        @pl.when(k + 1 <= NBB - 1)
        def _():
            for c in rd_copies(k + 1, sb_n, db_n, wb_n, semr_n):
                c.start()
        for c in rd_copies(k, sb_c, db_c, wb_c, semr_c):
            c.wait()

        @pl.when(k >= 2)
        def _():
            wr_copy(k - 2, pk_c, semw_c).wait()
        compute(sb_c, db_c, wb_c, pk_c)
        wr_copy(k, pk_c, semw_c).start()

    def pair_body(j, carry):
        blk(2 * j, bufs0, bufs1)
        blk(2 * j + 1, bufs1, bufs0)
        return carry
    for c in rd_copies(0, sb0, db0, wb0, semr0):
        c.start()
    lax.fori_loop(0, NBB // 2, pair_body, 0)
    blk(NBB - 1, bufs0, bufs1)

    wr_copy(NBB - 2, pk1, semw1).wait()
    wr_copy(NBB - 1, pk0, semw0).wait()

    pltpu.sync_copy(den_v, den_sh.at[rix_v], add=True)
    plsc.subcore_barrier()
    rpt = DEN_R // NS
    nw16 = DEN_R // 16

    @pl.when(sid < nw16)
    def _():
        pltpu.sync_copy(den_sh.at[pl.ds(sid * 16, 16)],
                        den_hbm.at[cid, pl.ds(sid * 16, 16)])


BN = 2000
NBN = EPT // BN


@functools.partial(
    pl.kernel, mesh=_mesh,
    compiler_params=_sc_params,
    out_type=jax.ShapeDtypeStruct((E * 8,), jnp.int32),
    scratch_types=[
        pltpu.VMEM((DEN_R, 128), jnp.float32),
        pltpu.VMEM((DEN_R, 128), jnp.float32),
        pltpu.VMEM((BN * 8,), jnp.int32),
        pltpu.VMEM((BN * 8,), jnp.int32),
        pltpu.SemaphoreType.DMA,
        pltpu.SemaphoreType.DMA,
        pltpu.SemaphoreType.DMA,
        pltpu.SemaphoreType.DMA,
    ])
def _normalize(pkin_hbm, den_hbm, pkout_hbm,
               rec_v, den1_v, pk0, pk1, semr0, semr1, semw0, semw1):
    cid = lax.axis_index("c")
    sid = lax.axis_index("s")
    wid = sid * NC + cid
    base = wid * EPT

    pltpu.sync_copy(den_hbm.at[0], rec_v)
    pltpu.sync_copy(den_hbm.at[1], den1_v)
    iota = lax.iota(jnp.int32, L)

    def rec_body(i, carry):
        r = i // 8
        c = (i % 8) * L
        rec_v[r, pl.ds(c, L)] = 1.0 / (
            rec_v[r, pl.ds(c, L)] + den1_v[r, pl.ds(c, L)] + 1e-16)
        return carry
    lax.fori_loop(0, DEN_R * 8, rec_body, 0, unroll=4)

    def rd_copy(k, pk, sem):
        return pltpu.make_async_copy(
            pkin_hbm.at[pl.ds((base + k * BN) * 8, BN * 8)], pk, sem)

    def wr_copy(k, pk, sem):
        return pltpu.make_async_copy(
            pk, pkout_hbm.at[pl.ds((base + k * BN) * 8, BN * 8)], sem)

    def patch(pk):
        def al_body(g, c2):
            jl = iota + g * L
            e_of_j = jl // 3
            idx = e_of_j * 8 + 3 + jl % 3
            pv = plsc.bitcast(plsc.load_gather(pk, [idx]), jnp.float32)
            dv = plsc.load_gather(pk, [e_of_j * 8 + 1])
            ridx = dv * 3 + jl % 3
            rv = plsc.load_gather(
                rec_v, [lax.shift_right_logical(ridx, 7),
                        jnp.bitwise_and(ridx, 127)])
            plsc.store_scatter(pk, [idx], plsc.bitcast(pv * rv, jnp.int32))
            return c2
        lax.fori_loop(0, (BN * 3) // L, al_body, 0)

    bufs0 = (pk0, semr0, semw0)
    bufs1 = (pk1, semr1, semw1)

    def blk(k, cur, nxt):
        pk_c, semr_c, semw_c = cur
        pk_n, semr_n, semw_n = nxt

        @pl.when(jnp.logical_and(k >= 1, k + 1 <= NBN - 1))
        def _():
            wr_copy(k - 1, pk_n, semw_n).wait()

        @pl.when(k + 1 <= NBN - 1)
        def _():
            rd_copy(k + 1, pk_n, semr_n).start()
        rd_copy(k, pk_c, semr_c).wait()
        patch(pk_c)
        wr_copy(k, pk_c, semw_c).start()

    def pair_body(j, carry):
        blk(2 * j, bufs0, bufs1)
        blk(2 * j + 1, bufs1, bufs0)
        return carry
    rd_copy(0, pk0, semr0).start()
    lax.fori_loop(0, NBN // 2, pair_body, 0)
    blk(NBN - 1, bufs0, bufs1)

    wr_copy(NBN - 2, pk1, semw1).wait()
    wr_copy(NBN - 1, pk0, semw0).wait()


@functools.partial(
    pl.kernel, mesh=_mesh,
    compiler_params=pltpu.CompilerParams(needs_layout_passes=False,
                                         use_tc_tiling_on_sc=False),
    out_type=jax.ShapeDtypeStruct((NC, N, D), jnp.float32),
    scratch_types=[
        pltpu.VMEM((BC * 8 + L,), jnp.int32),
        pltpu.VMEM((BC * 8 + L,), jnp.int32),
        pltpu.VMEM((BC,), jnp.int32),
        pltpu.VMEM((BC,), jnp.int32),
        pltpu.VMEM((BC,), jnp.int32),
        pltpu.VMEM((BC,), jnp.int32),
        pltpu.VMEM((BC,), jnp.int32),
        pltpu.VMEM((BC,), jnp.int32),
        pltpu.VMEM((TAIL,), jnp.int32),
        pltpu.VMEM((TAIL,), jnp.int32),
        pltpu.VMEM((TAIL,), jnp.int32),
        pltpu.VMEM((BC, HROW), jnp.int32),
        pltpu.VMEM((BC, HROW), jnp.int32),
        pltpu.VMEM((BC, D), jnp.float32),
        pltpu.VMEM((BC, D), jnp.float32),
        pltpu.VMEM_SHARED((N, D), jnp.float32),
        pltpu.SemaphoreType.DMA,
        pltpu.SemaphoreType.DMA,
        pltpu.SemaphoreType.DMA,
        pltpu.SemaphoreType.DMA,
        pltpu.SemaphoreType.DMA,
        pltpu.SemaphoreType.DMA,
    ])
def _phase_c(pk_hbm, h_hbm, tab_hbm, out_hbm,
             pk0, pk1, sb0, db0, wb0, sb1, db1, wb1,
             sbt, dbt, wbt, hr0, hr1, ew_v, msg_v,
             acc_sh, semp0, semp1, semh0, semh1, seme, sems):
    cid = lax.axis_index("c")
    sid = lax.axis_index("s")
    wid = sid * NC + cid
    base = wid * EPT

    zeros = jnp.zeros((L,), jnp.float32)

    def zero_body(i, carry):
        r = i // (D // L)
        c = i % (D // L)
        msg_v[r, pl.ds(c * L, L)] = zeros
        return carry
    lax.fori_loop(0, BC * (D // L), zero_body, 0)
    nfull = N // BC
    for j in range((nfull + NS) // NS):
        chunk = sid + j * NS

        @pl.when(chunk < nfull)
        def _():
            pltpu.sync_copy(msg_v, acc_sh.at[pl.ds(chunk * BC, BC)])

        @pl.when(chunk == nfull)
        def _():
            pltpu.sync_copy(msg_v.at[pl.ds(0, N - nfull * BC)],
                            acc_sh.at[pl.ds(nfull * BC, N - nfull * BC)])
    plsc.subcore_barrier()

    iota = lax.iota(jnp.int32, L)
    ilv = plsc.PackFormat.INTERLEAVED

    def build_sw(pk, sb, wb):
        for g in range(BC // L):
            lane = iota + g * L
            l8 = lane * 8
            sb[pl.ds(g * L, L)] = plsc.load_gather(pk, [l8])
            wb[pl.ds(g * L, L)] = plsc.load_gather(pk, [l8 + 2])

    def build_d(pk, db):
        for g in range(BC // L):
            l8 = (iota + g * L) * 8
            db[pl.ds(g * L, L)] = plsc.load_gather(pk, [l8 + 1])

    def compute(pk, hr, n):
        def edge_body(e, c2):
            av = plsc.bitcast(pk[pl.ds(e * 8 + 3, L)], jnp.float32)
            a0 = av[0]
            a1 = av[1]
            a2 = av[2]
            for g in range(D // 32):
                h0 = plsc.bitcast(hr[e, pl.ds(g * L, L)], jnp.bfloat16)
                h1 = plsc.bitcast(hr[e, pl.ds(64 + g * L, L)], jnp.bfloat16)
                h2 = plsc.bitcast(hr[e, pl.ds(128 + g * L, L)],
                                  jnp.bfloat16)
                h0a, h0b = plsc.unpack(h0, format=ilv)
                h1a, h1b = plsc.unpack(h1, format=ilv)
                h2a, h2b = plsc.unpack(h2, format=ilv)
                ea = ew_v[e, pl.ds(g * 32, L)]
                eb = ew_v[e, pl.ds(g * 32 + L, L)]
                ma = (a0 * h0a + a1 * h1a + a2 * h2a) * ea
                mb = (a0 * h0b + a1 * h1b + a2 * h2b) * eb
                msg_v[e, pl.ds(g * 32, L)] = ma
                msg_v[e, pl.ds(g * 32 + L, L)] = mb
            return c2
        lax.fori_loop(0, n, edge_body, 0, unroll=4)

    def pk_slice(k):
        return pk_hbm.at[pl.ds((base + k * BC) * 8, BC * 8)]

    pltpu.sync_copy(pk_slice(0), pk0.at[pl.ds(0, BC * 8)])
    build_sw(pk0, sb0, wb0)
    build_d(pk0, db0)
    pltpu.async_copy(h_hbm.at[sb0], hr0, semh0)
    pltpu.async_copy(tab_hbm.at[wb0], ew_v, seme)
    pltpu.async_copy(pk_slice(1), pk1.at[pl.ds(0, BC * 8)], semp1)

    bufs0 = (pk0, sb0, db0, wb0, hr0, semp0, semh0)
    bufs1 = (pk1, sb1, db1, wb1, hr1, semp1, semh1)

    def blk(k, cur, nxt):
        pk_c, sb_c, db_c, wb_c, hr_c, semp_c, semh_c = cur
        pk_n, sb_n, db_n, wb_n, hr_n, semp_n, semh_n = nxt

        @pl.when(k + 1 <= NBLK - 1)
        def _():
            pltpu.make_async_copy(pk_slice(k + 1), pk_n.at[pl.ds(0, BC * 8)],
                                  semp_n).wait()
            build_sw(pk_n, sb_n, wb_n)
            pltpu.async_copy(h_hbm.at[sb_n], hr_n, semh_n)

        @pl.when(k >= 1)
        def _():
            pltpu.make_async_copy(msg_v, acc_sh.at[db_n], sems).wait()

        @pl.when(k + 1 <= NBLK - 1)
        def _():
            build_d(pk_n, db_n)

        pltpu.make_async_copy(h_hbm.at[sb_c], hr_c, semh_c).wait()
        pltpu.make_async_copy(tab_hbm.at[wb_c], ew_v, seme).wait()
        compute(pk_c, hr_c, BC)
        pltpu.async_copy(msg_v, acc_sh.at[db_c], sems, add=True)

        @pl.when(k + 2 <= NBLK - 1)
        def _():
            pltpu.async_copy(pk_slice(k + 2), pk_c.at[pl.ds(0, BC * 8)],
                             semp_c)

        @pl.when(k + 1 <= NBLK - 1)
        def _():
            pltpu.async_copy(tab_hbm.at[wb_n], ew_v, seme)

    def pair_body(j, carry):
        blk(2 * j, bufs0, bufs1)
        blk(2 * j + 1, bufs1, bufs0)
        return carry
    lax.fori_loop(0, NBLK // 2, pair_body, 0)

    pltpu.make_async_copy(msg_v, acc_sh.at[db1], sems).wait()

    toff = base + NBLK * BC
    pltpu.sync_copy(pk_hbm.at[pl.ds(toff * 8, TAIL * 8)],
                    pk0.at[pl.ds(0, TAIL * 8)])
    l8 = iota * 8
    sbt[pl.ds(0, L)] = plsc.load_gather(pk0, [l8])
    dbt[pl.ds(0, L)] = plsc.load_gather(pk0, [l8 + 1])
    wbt[pl.ds(0, L)] = plsc.load_gather(pk0, [l8 + 2])
    pltpu.async_copy(h_hbm.at[sbt], hr0.at[pl.ds(0, TAIL)], semh0).wait()
    pltpu.async_copy(tab_hbm.at[wbt], ew_v.at[pl.ds(0, TAIL)], seme).wait()
    compute(pk0, hr0, TAIL)
    pltpu.sync_copy(msg_v.at[pl.ds(0, TAIL)], acc_sh.at[dbt], add=True)

    plsc.subcore_barrier()
    r1 = 632

    @pl.when(sid < NS - 1)
    def _():
        pltpu.sync_copy(acc_sh.at[pl.ds(sid * r1, r1)],
                        out_hbm.at[cid, pl.ds(sid * r1, r1)])

    @pl.when(sid == NS - 1)
    def _():
        pltpu.sync_copy(acc_sh.at[pl.ds((NS - 1) * r1, N - (NS - 1) * r1)],
                        out_hbm.at[cid, pl.ds((NS - 1) * r1,
                                              N - (NS - 1) * r1)])


def _combine_body(parts_ref, out_ref):
    out_ref[...] = (parts_ref[0] + parts_ref[1]) * (1.0 / NH)


def _combine(parts):
    return pl.pallas_call(
        _combine_body,
        out_shape=jax.ShapeDtypeStruct((N, D), jnp.float32),
    )(parts)


def kernel(x, edge_index, edge_weight, W_lin, edge_table, W_heads, a_src,
           a_dst):
    src = edge_index[0].astype(jnp.int32)
    dst = edge_index[1].astype(jnp.int32)
    w = edge_weight.astype(jnp.int32)
    slo = jnp.asarray(_SLO)
    shi = jnp.asarray(_SHI)
    h_i32, asd = _dense(x, W_lin, W_heads, a_src, a_dst, slo, shi)
    pk_raw, den_parts = _phase_b(src, dst, w, asd.reshape(-1))
    packed = _normalize(pk_raw, den_parts)
    parts = _phase_c(packed, h_i32, edge_table)
    return _combine(parts)


# speedup vs baseline: 44.0336x; 1.0436x over previous
import functools

import jax
import jax.numpy as jnp
import numpy as np
from jax import lax
from jax.experimental import pallas as pl
from jax.experimental.pallas import tpu as pltpu
from jax.experimental.pallas import tpu_sc as plsc

N = 10000
E = 320000
D = 128
NH = 3
NEG_SLOPE = 0.2
V = 22754

NC = 2
NS = 16
L = 16
NW = NC * NS
EPT = E // NW
BB = 400
BC = 64
NBLK = EPT // BC
TAIL = EPT - NBLK * BC
HROW = 192

_mesh = plsc.VectorSubcoreMesh(core_axis_name="c", subcore_axis_name="s")
_sc_params = pltpu.CompilerParams(needs_layout_passes=False)

_SLO = np.zeros((D, D // 2), np.float32)
_SHI = np.zeros((D, D // 2), np.float32)
for _g in range(D // 32):
    for _i in range(16):
        _SLO[_g * 32 + _i, _g * 16 + _i] = 1.0
        _SHI[_g * 32 + 16 + _i, _g * 16 + _i] = 1.0


def _bf16_bits(x):
    r = x.astype(jnp.bfloat16).astype(jnp.float32)
    return lax.bitcast_convert_type(r, jnp.int32)


def _dense_body(x_ref, wlin_ref, wh_ref, asrc_ref, adst_ref, slo_ref,
                shi_ref, h_ref, asd_ref):
    x1 = jnp.dot(x_ref[...], wlin_ref[...], preferred_element_type=jnp.float32)
    slo = slo_ref[...]
    shi = shi_ref[...]
    cols = []
    words = []
    for h in range(NH):
        wh = wh_ref[h, :, :]
        hp = jnp.dot(x1, wh, preferred_element_type=jnp.float32)
        lo = jnp.dot(hp, slo, preferred_element_type=jnp.float32)
        hi = jnp.dot(hp, shi, preferred_element_type=jnp.float32)
        words.append(jnp.bitwise_or(
            lax.shift_right_logical(_bf16_bits(lo), 16),
            jnp.bitwise_and(_bf16_bits(hi), jnp.int32(-65536))))
        vs = jnp.dot(wh, asrc_ref[h, :].reshape(D, 1),
                     preferred_element_type=jnp.float32)
        vd = jnp.dot(wh, adst_ref[h, :].reshape(D, 1),
                     preferred_element_type=jnp.float32)
        cols.append((jnp.dot(x1, vs, preferred_element_type=jnp.float32),
                     jnp.dot(x1, vd, preferred_element_type=jnp.float32)))
    h_ref[...] = jnp.concatenate(words, axis=1)
    asd_ref[...] = jnp.concatenate(
        [cols[0][0], cols[1][0], cols[2][0],
         cols[0][1], cols[1][1], cols[2][1]], axis=1)


def _dense(x, w_lin, w_heads, a_src, a_dst, slo, shi):
    return pl.pallas_call(
        _dense_body,
        out_shape=[jax.ShapeDtypeStruct((N, HROW), jnp.int32),
                   jax.ShapeDtypeStruct((N, 6), jnp.float32)],
    )(x, w_lin, w_heads, a_src, a_dst, slo, shi)


def _packtab_body(tab_ref, slo_ref, shi_ref, out_ref):
    lo = jnp.dot(tab_ref[...], slo_ref[...],
                 preferred_element_type=jnp.float32)
    hi = jnp.dot(tab_ref[...], shi_ref[...],
                 preferred_element_type=jnp.float32)
    out_ref[...] = jnp.bitwise_or(
        lax.shift_right_logical(_bf16_bits(lo), 16),
        jnp.bitwise_and(_bf16_bits(hi), jnp.int32(-65536)))


def _pack_table(tab, slo, shi):
    blk = 4096
    grid = (V + blk - 1) // blk
    return pl.pallas_call(
        _packtab_body,
        grid=(grid,),
        in_specs=[pl.BlockSpec((blk, D), lambda i: (i, 0)),
                  pl.BlockSpec((D, D // 2), lambda i: (0, 0)),
                  pl.BlockSpec((D, D // 2), lambda i: (0, 0))],
        out_specs=pl.BlockSpec((blk, D // 2), lambda i: (i, 0)),
        out_shape=jax.ShapeDtypeStruct((V, D // 2), jnp.int32),
    )(tab, slo, shi)


NBB = EPT // BB


DEN_R = 240


@functools.partial(
    pl.kernel, mesh=_mesh,
    compiler_params=_sc_params,
    out_type=[jax.ShapeDtypeStruct((E * 8,), jnp.int32),
              jax.ShapeDtypeStruct((NC, DEN_R, 128), jnp.float32)],
    scratch_types=[
        pltpu.VMEM((N * 6,), jnp.float32),
        pltpu.VMEM((DEN_R, 128), jnp.float32),
        pltpu.VMEM((DEN_R,), jnp.int32),
        pltpu.VMEM((BB,), jnp.int32),
        pltpu.VMEM((BB,), jnp.int32),
        pltpu.VMEM((BB,), jnp.int32),
        pltpu.VMEM((BB,), jnp.int32),
        pltpu.VMEM((BB,), jnp.int32),
        pltpu.VMEM((BB,), jnp.int32),
        pltpu.VMEM((BB * 8,), jnp.int32),
        pltpu.VMEM((BB * 8,), jnp.int32),
        pltpu.VMEM_SHARED((DEN_R, 128), jnp.float32),
        pltpu.SemaphoreType.DMA,
        pltpu.SemaphoreType.DMA,
        pltpu.SemaphoreType.DMA,
        pltpu.SemaphoreType.DMA,
    ])
def _phase_b(src_hbm, dst_hbm, w_hbm, asd_hbm, pk_hbm, den_hbm,
             asd_v, den_v, rix_v, sb0, db0, wb0, sb1, db1, wb1, pk0, pk1,
             den_sh, semr0, semr1, semw0, semw1):
    cid = lax.axis_index("c")
    sid = lax.axis_index("s")
    wid = sid * NC + cid
    base = wid * EPT

    pltpu.sync_copy(asd_hbm, asd_v)

    zeros = jnp.zeros((L,), jnp.float32)
    iota = lax.iota(jnp.int32, L)

    def zero_body(i, carry):
        den_v[i // 8, pl.ds((i % 8) * L, L)] = zeros
        return carry
    lax.fori_loop(0, DEN_R * 8, zero_body, 0)

    def rix_body(g, carry):
        rix_v[pl.ds(g * L, L)] = iota + g * L
        return carry
    lax.fori_loop(0, DEN_R // L, rix_body, 0)

    pltpu.sync_copy(den_v.at[pl.ds(0, DEN_R // NS)],
                    den_sh.at[pl.ds(sid * (DEN_R // NS), DEN_R // NS)])
    plsc.subcore_barrier()

    def rd_copies(k, sb, db, wb, sem):
        off = base + k * BB
        return (pltpu.make_async_copy(src_hbm.at[pl.ds(off, BB)], sb, sem),
                pltpu.make_async_copy(dst_hbm.at[pl.ds(off, BB)], db, sem),
                pltpu.make_async_copy(w_hbm.at[pl.ds(off, BB)], wb, sem))

    def wr_copy(k, pk, sem):
        return pltpu.make_async_copy(
            pk, pk_hbm.at[pl.ds((base + k * BB) * 8, BB * 8)], sem)

    def compute(sb, db, wb, pk):
        def grp_body(i, c2):
            sv = sb[pl.ds(i * L, L)]
            dv = db[pl.ds(i * L, L)]
            wv = wb[pl.ds(i * L, L)]
            s6 = sv * 6
            d6 = dv * 6
            d3 = dv * 3
            l8 = (iota + i * L) * 8
            plsc.store_scatter(pk, [l8], sv)
            plsc.store_scatter(pk, [l8 + 1], dv)
            plsc.store_scatter(pk, [l8 + 2], wv)
            for h in range(NH):
                va = plsc.load_gather(asd_v, [s6 + h])
                vb = plsc.load_gather(asd_v, [d6 + (3 + h)])
                e = va + vb
                e = jnp.where(e >= 0.0, e, e * NEG_SLOPE)
                p = jnp.exp(e)
                idx = d3 + h
                plsc.addupdate_scatter(
                    den_v, [lax.shift_right_logical(idx, 7),
                            jnp.bitwise_and(idx, 127)], p)
                plsc.store_scatter(pk, [l8 + 3 + h],
                                   plsc.bitcast(p, jnp.int32))
            return c2
        lax.fori_loop(0, BB // L, grp_body, 0)

    bufs0 = (sb0, db0, wb0, pk0, semr0, semw0)
    bufs1 = (sb1, db1, wb1, pk1, semr1, semw1)

    def blk(k, cur, nxt):
        sb_c, db_c, wb_c, pk_c, semr_c, semw_c = cur
        sb_n, db_n, wb_n, pk_n, semr_n, semw_n = nxt

        @pl.when(k + 1 <= NBB - 1)
        def _():
            for c in rd_copies(k + 1, sb_n, db_n, wb_n, semr_n):
                c.start()
        for c in rd_copies(k, sb_c, db_c, wb_c, semr_c):
            c.wait()

        @pl.when(k >= 2)
        def _():
            wr_copy(k - 2, pk_c, semw_c).wait()
        compute(sb_c, db_c, wb_c, pk_c)
        wr_copy(k, pk_c, semw_c).start()

    def pair_body(j, carry):
        blk(2 * j, bufs0, bufs1)
        blk(2 * j + 1, bufs1, bufs0)
        return carry
    for c in rd_copies(0, sb0, db0, wb0, semr0):
        c.start()
    lax.fori_loop(0, NBB // 2, pair_body, 0)
    blk(NBB - 1, bufs0, bufs1)

    wr_copy(NBB - 2, pk1, semw1).wait()
    wr_copy(NBB - 1, pk0, semw0).wait()

    pltpu.sync_copy(den_v, den_sh.at[rix_v], add=True)
    plsc.subcore_barrier()
    rpt = DEN_R // NS
    nw16 = DEN_R // 16

    @pl.when(sid < nw16)
    def _():
        pltpu.sync_copy(den_sh.at[pl.ds(sid * 16, 16)],
                        den_hbm.at[cid, pl.ds(sid * 16, 16)])


BN = 2000
NBN = EPT // BN


@functools.partial(
    pl.kernel, mesh=_mesh,
    compiler_params=_sc_params,
    out_type=jax.ShapeDtypeStruct((E * 8,), jnp.int32),
    scratch_types=[
        pltpu.VMEM((DEN_R, 128), jnp.float32),
        pltpu.VMEM((DEN_R, 128), jnp.float32),
        pltpu.VMEM((BN * 8,), jnp.int32),
        pltpu.VMEM((BN * 8,), jnp.int32),
        pltpu.SemaphoreType.DMA,
        pltpu.SemaphoreType.DMA,
        pltpu.SemaphoreType.DMA,
        pltpu.SemaphoreType.DMA,
    ])
def _normalize(pkin_hbm, den_hbm, pkout_hbm,
               rec_v, den1_v, pk0, pk1, semr0, semr1, semw0, semw1):
    cid = lax.axis_index("c")
    sid = lax.axis_index("s")
    wid = sid * NC + cid
    base = wid * EPT

    pltpu.sync_copy(den_hbm.at[0], rec_v)
    pltpu.sync_copy(den_hbm.at[1], den1_v)
    iota = lax.iota(jnp.int32, L)

    def rec_body(i, carry):
        r = i // 8
        c = (i % 8) * L
        rec_v[r, pl.ds(c, L)] = 1.0 / (
            rec_v[r, pl.ds(c, L)] + den1_v[r, pl.ds(c, L)] + 1e-16)
        return carry
    lax.fori_loop(0, DEN_R * 8, rec_body, 0, unroll=4)

    def rd_copy(k, pk, sem):
        return pltpu.make_async_copy(
            pkin_hbm.at[pl.ds((base + k * BN) * 8, BN * 8)], pk, sem)

    def wr_copy(k, pk, sem):
        return pltpu.make_async_copy(
            pk, pkout_hbm.at[pl.ds((base + k * BN) * 8, BN * 8)], sem)

    def patch(pk):
        def al_body(g, c2):
            jl = iota + g * L
            e_of_j = jl // 3
            idx = e_of_j * 8 + 3 + jl % 3
            pv = plsc.bitcast(plsc.load_gather(pk, [idx]), jnp.float32)
            dv = plsc.load_gather(pk, [e_of_j * 8 + 1])
            ridx = dv * 3 + jl % 3
            rv = plsc.load_gather(
                rec_v, [lax.shift_right_logical(ridx, 7),
                        jnp.bitwise_and(ridx, 127)])
            plsc.store_scatter(pk, [idx], plsc.bitcast(pv * rv, jnp.int32))
            return c2
        lax.fori_loop(0, (BN * 3) // L, al_body, 0)

    bufs0 = (pk0, semr0, semw0)
    bufs1 = (pk1, semr1, semw1)

    def blk(k, cur, nxt):
        pk_c, semr_c, semw_c = cur
        pk_n, semr_n, semw_n = nxt

        @pl.when(jnp.logical_and(k >= 1, k + 1 <= NBN - 1))
        def _():
            wr_copy(k - 1, pk_n, semw_n).wait()

        @pl.when(k + 1 <= NBN - 1)
        def _():
            rd_copy(k + 1, pk_n, semr_n).start()
        rd_copy(k, pk_c, semr_c).wait()
        patch(pk_c)
        wr_copy(k, pk_c, semw_c).start()

    def pair_body(j, carry):
        blk(2 * j, bufs0, bufs1)
        blk(2 * j + 1, bufs1, bufs0)
        return carry
    rd_copy(0, pk0, semr0).start()
    lax.fori_loop(0, NBN // 2, pair_body, 0)
    blk(NBN - 1, bufs0, bufs1)

    wr_copy(NBN - 2, pk1, semw1).wait()
    wr_copy(NBN - 1, pk0, semw0).wait()


@functools.partial(
    pl.kernel, mesh=_mesh,
    compiler_params=pltpu.CompilerParams(needs_layout_passes=False,
                                         use_tc_tiling_on_sc=False),
    out_type=jax.ShapeDtypeStruct((NC, N, D), jnp.float32),
    scratch_types=[
        pltpu.VMEM((BC * 8 + L,), jnp.int32),
        pltpu.VMEM((BC * 8 + L,), jnp.int32),
        pltpu.VMEM((BC,), jnp.int32),
        pltpu.VMEM((BC,), jnp.int32),
        pltpu.VMEM((BC,), jnp.int32),
        pltpu.VMEM((BC,), jnp.int32),
        pltpu.VMEM((BC,), jnp.int32),
        pltpu.VMEM((BC,), jnp.int32),
        pltpu.VMEM((TAIL,), jnp.int32),
        pltpu.VMEM((TAIL,), jnp.int32),
        pltpu.VMEM((TAIL,), jnp.int32),
        pltpu.VMEM((BC, HROW), jnp.int32),
        pltpu.VMEM((BC, HROW), jnp.int32),
        pltpu.VMEM((BC, D // 2), jnp.int32),
        pltpu.VMEM((BC, D), jnp.float32),
        pltpu.VMEM_SHARED((N, D), jnp.float32),
        pltpu.SemaphoreType.DMA,
        pltpu.SemaphoreType.DMA,
        pltpu.SemaphoreType.DMA,
        pltpu.SemaphoreType.DMA,
        pltpu.SemaphoreType.DMA,
        pltpu.SemaphoreType.DMA,
    ])
def _phase_c(pk_hbm, h_hbm, tab_hbm, out_hbm,
             pk0, pk1, sb0, db0, wb0, sb1, db1, wb1,
             sbt, dbt, wbt, hr0, hr1, ew_v, msg_v,
             acc_sh, semp0, semp1, semh0, semh1, seme, sems):
    cid = lax.axis_index("c")
    sid = lax.axis_index("s")
    wid = sid * NC + cid
    base = wid * EPT

    zeros = jnp.zeros((L,), jnp.float32)

    def zero_body(i, carry):
        r = i // (D // L)
        c = i % (D // L)
        msg_v[r, pl.ds(c * L, L)] = zeros
        return carry
    lax.fori_loop(0, BC * (D // L), zero_body, 0)
    nfull = N // BC
    for j in range((nfull + NS) // NS):
        chunk = sid + j * NS

        @pl.when(chunk < nfull)
        def _():
            pltpu.sync_copy(msg_v, acc_sh.at[pl.ds(chunk * BC, BC)])

        @pl.when(chunk == nfull)
        def _():
            pltpu.sync_copy(msg_v.at[pl.ds(0, N - nfull * BC)],
                            acc_sh.at[pl.ds(nfull * BC, N - nfull * BC)])
    plsc.subcore_barrier()

    iota = lax.iota(jnp.int32, L)
    ilv = plsc.PackFormat.INTERLEAVED

    def build_sw(pk, sb, wb):
        for g in range(BC // L):
            lane = iota + g * L
            l8 = lane * 8
            sb[pl.ds(g * L, L)] = plsc.load_gather(pk, [l8])
            wb[pl.ds(g * L, L)] = plsc.load_gather(pk, [l8 + 2])

    def build_d(pk, db):
        for g in range(BC // L):
            l8 = (iota + g * L) * 8
            db[pl.ds(g * L, L)] = plsc.load_gather(pk, [l8 + 1])

    def compute(pk, hr, n):
        def edge_body(e, c2):
            av = plsc.bitcast(pk[pl.ds(e * 8 + 3, L)], jnp.float32)
            a0 = av[0]
            a1 = av[1]
            a2 = av[2]
            for g in range(D // 32):
                h0 = plsc.bitcast(hr[e, pl.ds(g * L, L)], jnp.bfloat16)
                h1 = plsc.bitcast(hr[e, pl.ds(64 + g * L, L)], jnp.bfloat16)
                h2 = plsc.bitcast(hr[e, pl.ds(128 + g * L, L)],
                                  jnp.bfloat16)
                h0a, h0b = plsc.unpack(h0, format=ilv)
                h1a, h1b = plsc.unpack(h1, format=ilv)
                h2a, h2b = plsc.unpack(h2, format=ilv)
                ev = plsc.bitcast(ew_v[e, pl.ds(g * L, L)], jnp.bfloat16)
                ea, eb = plsc.unpack(ev, format=ilv)
                ma = (a0 * h0a + a1 * h1a + a2 * h2a) * ea
                mb = (a0 * h0b + a1 * h1b + a2 * h2b) * eb
                msg_v[e, pl.ds(g * 32, L)] = ma
                msg_v[e, pl.ds(g * 32 + L, L)] = mb
            return c2
        lax.fori_loop(0, n, edge_body, 0, unroll=4)

    def pk_slice(k):
        return pk_hbm.at[pl.ds((base + k * BC) * 8, BC * 8)]

    pltpu.sync_copy(pk_slice(0), pk0.at[pl.ds(0, BC * 8)])
    build_sw(pk0, sb0, wb0)
    build_d(pk0, db0)
    pltpu.async_copy(h_hbm.at[sb0], hr0, semh0)
    pltpu.async_copy(tab_hbm.at[wb0], ew_v, seme)
    pltpu.async_copy(pk_slice(1), pk1.at[pl.ds(0, BC * 8)], semp1)

    bufs0 = (pk0, sb0, db0, wb0, hr0, semp0, semh0)
    bufs1 = (pk1, sb1, db1, wb1, hr1, semp1, semh1)

    def blk(k, cur, nxt):
        pk_c, sb_c, db_c, wb_c, hr_c, semp_c, semh_c = cur
        pk_n, sb_n, db_n, wb_n, hr_n, semp_n, semh_n = nxt

        @pl.when(k + 1 <= NBLK - 1)
        def _():
            pltpu.make_async_copy(pk_slice(k + 1), pk_n.at[pl.ds(0, BC * 8)],
                                  semp_n).wait()
            build_sw(pk_n, sb_n, wb_n)
            pltpu.async_copy(h_hbm.at[sb_n], hr_n, semh_n)

        @pl.when(k >= 1)
        def _():
            pltpu.make_async_copy(msg_v, acc_sh.at[db_n], sems).wait()

        @pl.when(k + 1 <= NBLK - 1)
        def _():
            build_d(pk_n, db_n)

        pltpu.make_async_copy(h_hbm.at[sb_c], hr_c, semh_c).wait()
        pltpu.make_async_copy(tab_hbm.at[wb_c], ew_v, seme).wait()
        compute(pk_c, hr_c, BC)
        pltpu.async_copy(msg_v, acc_sh.at[db_c], sems, add=True)

        @pl.when(k + 2 <= NBLK - 1)
        def _():
            pltpu.async_copy(pk_slice(k + 2), pk_c.at[pl.ds(0, BC * 8)],
                             semp_c)

        @pl.when(k + 1 <= NBLK - 1)
        def _():
            pltpu.async_copy(tab_hbm.at[wb_n], ew_v, seme)

    def pair_body(j, carry):
        blk(2 * j, bufs0, bufs1)
        blk(2 * j + 1, bufs1, bufs0)
        return carry
    lax.fori_loop(0, NBLK // 2, pair_body, 0)

    pltpu.make_async_copy(msg_v, acc_sh.at[db1], sems).wait()

    toff = base + NBLK * BC
    pltpu.sync_copy(pk_hbm.at[pl.ds(toff * 8, TAIL * 8)],
                    pk0.at[pl.ds(0, TAIL * 8)])
    l8 = iota * 8
    sbt[pl.ds(0, L)] = plsc.load_gather(pk0, [l8])
    dbt[pl.ds(0, L)] = plsc.load_gather(pk0, [l8 + 1])
    wbt[pl.ds(0, L)] = plsc.load_gather(pk0, [l8 + 2])
    pltpu.async_copy(h_hbm.at[sbt], hr0.at[pl.ds(0, TAIL)], semh0).wait()
    pltpu.async_copy(tab_hbm.at[wbt], ew_v.at[pl.ds(0, TAIL)], seme).wait()
    compute(pk0, hr0, TAIL)
    pltpu.sync_copy(msg_v.at[pl.ds(0, TAIL)], acc_sh.at[dbt], add=True)

    plsc.subcore_barrier()
    r1 = 632

    @pl.when(sid < NS - 1)
    def _():
        pltpu.sync_copy(acc_sh.at[pl.ds(sid * r1, r1)],
                        out_hbm.at[cid, pl.ds(sid * r1, r1)])

    @pl.when(sid == NS - 1)
    def _():
        pltpu.sync_copy(acc_sh.at[pl.ds((NS - 1) * r1, N - (NS - 1) * r1)],
                        out_hbm.at[cid, pl.ds((NS - 1) * r1,
                                              N - (NS - 1) * r1)])


def _combine_body(parts_ref, out_ref):
    out_ref[...] = (parts_ref[0] + parts_ref[1]) * (1.0 / NH)


def _combine(parts):
    return pl.pallas_call(
        _combine_body,
        out_shape=jax.ShapeDtypeStruct((N, D), jnp.float32),
    )(parts)


def kernel(x, edge_index, edge_weight, W_lin, edge_table, W_heads, a_src,
           a_dst):
    src = edge_index[0].astype(jnp.int32)
    dst = edge_index[1].astype(jnp.int32)
    w = edge_weight.astype(jnp.int32)
    slo = jnp.asarray(_SLO)
    shi = jnp.asarray(_SHI)
    h_i32, asd = _dense(x, W_lin, W_heads, a_src, a_dst, slo, shi)
    tab_i32 = _pack_table(edge_table, slo, shi)
    pk_raw, den_parts = _phase_b(src, dst, w, asd.reshape(-1))
    packed = _normalize(pk_raw, den_parts)
    parts = _phase_c(packed, h_i32, tab_i32)
    return _combine(parts)
